# Initial kernel scaffold; baseline (speedup 1.0000x reference)
import jax
import jax.numpy as jnp
from jax.experimental import pallas as pl


def kernel(x, edge_index, kpts, pts_3d, W_pe1, b_pe1, W_pe2, b_pe2, W_res, b_res, W_l, b_l, W_r, b_r, W_e, att, b_conv, ln_g, ln_b, W_proj, b_proj):
    raise NotImplementedError("write your pallas kernel here")


# speedup vs baseline: 1.0075x; 1.0075x over previous
import functools

import jax
import jax.numpy as jnp
from jax.experimental import pallas as pl

N = 10000
E = 160000
D_IN = 256
D_CAT = 320
HID = 256
HEADS = 4
CH = 64

_BN = 1000


def _sigmoid(v):
    return 1.0 / (1.0 + jnp.exp(-v))


def _front_body(x_ref, kpts_ref, pts_ref, wpe1_ref, bpe1_ref, wpe2_ref,
                bpe2_ref, wres_ref, bres_ref, wl_ref, bl_ref, wr_ref, br_ref,
                xl_ref, xr_ref, ident_ref, nuv_ref):
    kpts = kpts_ref[...]
    nu = kpts[:, 0:1] * (1.0 / 1216.0)
    nv = kpts[:, 1:2] * (1.0 / 352.0)
    depth = pts_ref[:, 2:3]
    w1 = wpe1_ref[...]
    h = (nu * w1[0:1, :] + nv * w1[1:2, :] + depth * w1[2:3, :] + bpe1_ref[...])
    h = h * _sigmoid(h)
    pos = jnp.dot(h, wpe2_ref[...], preferred_element_type=jnp.float32) + bpe2_ref[...]
    x = x_ref[...]

    def lin(w_ref, b_ref):
        w = w_ref[...]
        return (jnp.dot(x, w[:D_IN, :], preferred_element_type=jnp.float32)
                + jnp.dot(pos, w[D_IN:, :], preferred_element_type=jnp.float32)
                + b_ref[...])

    xl_ref[...] = lin(wl_ref, bl_ref)
    xr_ref[...] = lin(wr_ref, br_ref)
    ident_ref[...] = lin(wres_ref, bres_ref)
    nuv_ref[...] = jnp.concatenate([nu, nv], axis=1)


def _front(x, kpts, pts_3d, W_pe1, b_pe1, W_pe2, b_pe2, W_res, b_res,
           W_l, b_l, W_r, b_r):
    grid = (N // _BN,)
    bspec = lambda shp: pl.BlockSpec(shp, lambda i: (i, 0))
    fspec = lambda shp: pl.BlockSpec(shp, lambda i: (0, 0))
    row = lambda d: pl.BlockSpec((d,), lambda i: (0,))
    return pl.pallas_call(
        _front_body,
        grid=grid,
        in_specs=[
            bspec((_BN, D_IN)), bspec((_BN, 2)), bspec((_BN, 3)),
            fspec((3, 32)), row(32), fspec((32, 64)), row(64),
            fspec((D_CAT, HID)), row(HID),
            fspec((D_CAT, HID)), row(HID),
            fspec((D_CAT, HID)), row(HID),
        ],
        out_specs=[
            bspec((_BN, HID)), bspec((_BN, HID)), bspec((_BN, HID)),
            bspec((_BN, 2)),
        ],
        out_shape=[
            jax.ShapeDtypeStruct((N, HID), jnp.float32),
            jax.ShapeDtypeStruct((N, HID), jnp.float32),
            jax.ShapeDtypeStruct((N, HID), jnp.float32),
            jax.ShapeDtypeStruct((N, 2), jnp.float32),
        ],
    )(x, kpts, pts_3d, W_pe1, b_pe1, W_pe2, b_pe2, W_res, b_res,
      W_l, b_l, W_r, b_r)


def _tail_body(acc_ref, ident_ref, bconv_ref, lng_ref, lnb_ref, wproj_ref,
               bproj_ref, out_ref):
    o = acc_ref[...] + bconv_ref[...]
    mu = jnp.mean(o, axis=-1, keepdims=True)
    var = jnp.mean((o - mu) ** 2, axis=-1, keepdims=True)
    o = (o - mu) / jnp.sqrt(var + 1e-5) * lng_ref[...] + lnb_ref[...]
    o = o * _sigmoid(o)
    o = o + ident_ref[...]
    out_ref[...] = (jnp.dot(o, wproj_ref[...], preferred_element_type=jnp.float32)
                    + bproj_ref[...])


def _tail(acc, ident, b_conv, ln_g, ln_b, W_proj, b_proj):
    grid = (N // _BN,)
    bspec = lambda shp: pl.BlockSpec(shp, lambda i: (i, 0))
    fspec = lambda shp: pl.BlockSpec(shp, lambda i: (0, 0))
    row = lambda d: pl.BlockSpec((d,), lambda i: (0,))
    return pl.pallas_call(
        _tail_body,
        grid=grid,
        in_specs=[
            bspec((_BN, HID)), bspec((_BN, HID)), row(HID),
            row(HID), row(HID), fspec((HID, 256)), row(256),
        ],
        out_specs=pl.BlockSpec((_BN, 256), lambda i: (i, 0)),
        out_shape=jax.ShapeDtypeStruct((N, 256), jnp.float32),
    )(acc, ident, b_conv, ln_g, ln_b, W_proj, b_proj)


def kernel(x, edge_index, kpts, pts_3d, W_pe1, b_pe1, W_pe2, b_pe2, W_res,
           b_res, W_l, b_l, W_r, b_r, W_e, att, b_conv, ln_g, ln_b, W_proj,
           b_proj):
    x_l, x_r, identity, norm_uv = _front(
        x, kpts, pts_3d, W_pe1, b_pe1, W_pe2, b_pe2, W_res, b_res,
        W_l, b_l, W_r, b_r)

    src = edge_index[0]
    dst = edge_index[1]
    rel_uv = norm_uv[dst] - norm_uv[src]
    dist = jnp.linalg.norm(rel_uv, axis=-1, keepdims=True)
    edge_attr = jnp.concatenate([rel_uv, dist], axis=-1)

    xl_h = x_l.reshape(N, HEADS, CH)
    xr_h = x_r.reshape(N, HEADS, CH)
    e_feat = (edge_attr @ W_e).reshape(-1, HEADS, CH)
    m = xl_h[src] + xr_h[dst] + e_feat
    m = jax.nn.leaky_relu(m, negative_slope=0.2)
    alpha = jnp.sum(m * att[None, :, :], axis=-1)
    amax = jax.ops.segment_max(alpha, dst, num_segments=N)
    amax = jnp.where(jnp.isfinite(amax), amax, 0.0)
    ex = jnp.exp(alpha - jax.lax.stop_gradient(amax)[dst])
    den = jax.ops.segment_sum(ex, dst, num_segments=N)
    alpha_sm = ex / (den[dst] + 1e-16)
    msg = xl_h[src] * alpha_sm[:, :, None]
    acc = jax.ops.segment_sum(msg, dst, num_segments=N).reshape(N, HEADS * CH)

    out = _tail(acc, identity, b_conv, ln_g, ln_b, W_proj, b_proj)
    return (out, alpha_sm, edge_attr)


# speedup vs baseline: 7.7632x; 7.7056x over previous
import functools

import jax
import jax.numpy as jnp
from jax import lax
from jax.experimental import pallas as pl
from jax.experimental.pallas import tpu as pltpu
from jax.experimental.pallas import tpu_sc as plsc

N = 10000
E = 160000
D_IN = 256
D_CAT = 320
HID = 256
HEADS = 4
CH = 64

_BN = 1000


def _sigmoid(v):
    return 1.0 / (1.0 + jnp.exp(-v))


def _front_body(x_ref, kpts_ref, pts_ref, wpe1_ref, bpe1_ref, wpe2_ref,
                bpe2_ref, wres_ref, bres_ref, wl_ref, bl_ref, wr_ref, br_ref,
                xl_ref, xr_ref, ident_ref, nuv_ref, xq0_ref, xq1_ref, xq2_ref, xq3_ref):
    kpts = kpts_ref[...]
    nu = kpts[:, 0:1] * (1.0 / 1216.0)
    nv = kpts[:, 1:2] * (1.0 / 352.0)
    depth = pts_ref[:, 2:3]
    w1 = wpe1_ref[...]
    h = (nu * w1[0:1, :] + nv * w1[1:2, :] + depth * w1[2:3, :] + bpe1_ref[...])
    h = h * _sigmoid(h)
    pos = jnp.dot(h, wpe2_ref[...], preferred_element_type=jnp.float32) + bpe2_ref[...]
    x = x_ref[...]

    def lin(w_ref, b_ref):
        w = w_ref[...]
        return (jnp.dot(x, w[:D_IN, :], preferred_element_type=jnp.float32)
                + jnp.dot(pos, w[D_IN:, :], preferred_element_type=jnp.float32)
                + b_ref[...])

    xl = lin(wl_ref, bl_ref)
    xl_ref[...] = xl
    xq0_ref[...] = xl[:, 0:CH]
    xq1_ref[...] = xl[:, CH:2 * CH]
    xq2_ref[...] = xl[:, 2 * CH:3 * CH]
    xq3_ref[...] = xl[:, 3 * CH:]
    xr_ref[...] = lin(wr_ref, br_ref)
    ident_ref[...] = lin(wres_ref, bres_ref)
    nuv_ref[...] = jnp.concatenate([nu, nv], axis=1)


def _front(x, kpts, pts_3d, W_pe1, b_pe1, W_pe2, b_pe2, W_res, b_res,
           W_l, b_l, W_r, b_r):
    grid = (N // _BN,)
    bspec = lambda shp: pl.BlockSpec(shp, lambda i: (i, 0))
    fspec = lambda shp: pl.BlockSpec(shp, lambda i: (0, 0))
    row = lambda d: pl.BlockSpec((d,), lambda i: (0,))
    return pl.pallas_call(
        _front_body,
        grid=grid,
        in_specs=[
            bspec((_BN, D_IN)), bspec((_BN, 2)), bspec((_BN, 3)),
            fspec((3, 32)), row(32), fspec((32, 64)), row(64),
            fspec((D_CAT, HID)), row(HID),
            fspec((D_CAT, HID)), row(HID),
            fspec((D_CAT, HID)), row(HID),
        ],
        out_specs=[
            bspec((_BN, HID)), bspec((_BN, HID)), bspec((_BN, HID)),
            bspec((_BN, 2)), bspec((_BN, CH)), bspec((_BN, CH)),
            bspec((_BN, CH)), bspec((_BN, CH)),
        ],
        out_shape=[
            jax.ShapeDtypeStruct((N, HID), jnp.float32),
            jax.ShapeDtypeStruct((N, HID), jnp.float32),
            jax.ShapeDtypeStruct((N, HID), jnp.float32),
            jax.ShapeDtypeStruct((N, 2), jnp.float32),
            jax.ShapeDtypeStruct((N, CH), jnp.float32),
            jax.ShapeDtypeStruct((N, CH), jnp.float32),
            jax.ShapeDtypeStruct((N, CH), jnp.float32),
            jax.ShapeDtypeStruct((N, CH), jnp.float32),
        ],
    )(x, kpts, pts_3d, W_pe1, b_pe1, W_pe2, b_pe2, W_res, b_res,
      W_l, b_l, W_r, b_r)


def _tail_body(acc_ref, ident_ref, bconv_ref, lng_ref, lnb_ref, wproj_ref,
               bproj_ref, out_ref):
    o = acc_ref[...] + bconv_ref[...]
    mu = jnp.mean(o, axis=-1, keepdims=True)
    var = jnp.mean((o - mu) ** 2, axis=-1, keepdims=True)
    o = (o - mu) / jnp.sqrt(var + 1e-5) * lng_ref[...] + lnb_ref[...]
    o = o * _sigmoid(o)
    o = o + ident_ref[...]
    out_ref[...] = (jnp.dot(o, wproj_ref[...], preferred_element_type=jnp.float32)
                    + bproj_ref[...])


def _tail(acc, ident, b_conv, ln_g, ln_b, W_proj, b_proj):
    grid = (N // _BN,)
    bspec = lambda shp: pl.BlockSpec(shp, lambda i: (i, 0))
    fspec = lambda shp: pl.BlockSpec(shp, lambda i: (0, 0))
    row = lambda d: pl.BlockSpec((d,), lambda i: (0,))
    return pl.pallas_call(
        _tail_body,
        grid=grid,
        in_specs=[
            bspec((_BN, HID)), bspec((_BN, HID)), row(HID),
            row(HID), row(HID), fspec((HID, 256)), row(256),
        ],
        out_specs=pl.BlockSpec((_BN, 256), lambda i: (i, 0)),
        out_shape=jax.ShapeDtypeStruct((N, 256), jnp.float32),
    )(acc, ident, b_conv, ln_g, ln_b, W_proj, b_proj)


_NC = 2
_NS = 16
_NW = _NC * _NS
_EPW = E // _NW
_KA = 128
_NCHUNKS = E // _KA


def _nsqrt(d2):
    i = plsc.bitcast(d2, jnp.int32)
    i = jnp.int32(0x5F3759DF) - lax.shift_right_logical(i, jnp.int32(1))
    y = plsc.bitcast(i, jnp.float32)
    for _ in range(3):
        y = y * (1.5 - 0.5 * d2 * y * y)
    return jnp.where(d2 > 0.0, d2 * y, 0.0)


def _pa_body(xl_hbm, xr_hbm, nuv_hbm, src_hbm, dst_hbm, we_hbm, att_hbm,
             alpha_hbm, ea_hbm, wmax_hbm,
             nuv_v, we_v, att_v, srcb, dstb, xlb, xrb, alb, eab, maxb, sem):
    w = lax.axis_index("s") * _NC + lax.axis_index("c")
    pltpu.sync_copy(nuv_hbm, nuv_v)
    pltpu.sync_copy(we_hbm, we_v)
    pltpu.sync_copy(att_hbm, att_v)
    iota16 = lax.iota(jnp.int32, 16)

    def process_chunk(cidx, wmaxs):
        base = cidx * _KA
        pltpu.sync_copy(src_hbm.at[pl.ds(base, _KA)], srcb)
        pltpu.sync_copy(dst_hbm.at[pl.ds(base, _KA)], dstb)
        pltpu.async_copy(xl_hbm.at[srcb], xlb, sem).wait()
        pltpu.async_copy(xr_hbm.at[dstb], xrb, sem).wait()

        def group_body(g, wm):
            sl = pl.ds(g * 16, 16)
            evec = iota16 + g * 16
            src2 = srcb[sl] * 2
            dst2 = dstb[sl] * 2
            u_s = plsc.load_gather(nuv_v, [src2])
            v_s = plsc.load_gather(nuv_v, [src2 + 1])
            u_d = plsc.load_gather(nuv_v, [dst2])
            v_d = plsc.load_gather(nuv_v, [dst2 + 1])
            ru = u_d - u_s
            rv = v_d - v_s
            dist = _nsqrt(ru * ru + rv * rv)
            eab[pl.ds(g * 16, 16)] = ru
            eab[pl.ds(_KA + g * 16, 16)] = rv
            eab[pl.ds(2 * _KA + g * 16, 16)] = dist
            wm_new = []
            for h in range(HEADS):
                def cbody(cb, acc, _h=h):
                    cbase = _h * CH + cb * 16
                    w0v = we_v[pl.ds(cbase, 16)]
                    w1v = we_v[pl.ds(HID + cbase, 16)]
                    w2v = we_v[pl.ds(2 * HID + cbase, 16)]
                    atv = att_v[pl.ds(_h * CH + cb * 16, 16)]
                    for j in range(16):
                        cf = jnp.full((16,), cbase + j, jnp.int32)
                        xl_c = plsc.load_gather(xlb, [evec, cf])
                        xr_c = plsc.load_gather(xrb, [evec, cf])
                        ef = ru * w0v[j] + rv * w1v[j] + dist * w2v[j]
                        m = xl_c + xr_c + ef
                        lk = jnp.maximum(m, 0.2 * m)
                        acc = acc + atv[j] * lk
                    return acc
                acc = lax.fori_loop(0, CH // 16, cbody,
                                    jnp.zeros((16,), jnp.float32))
                alb[pl.ds(h * _KA + g * 16, 16)] = acc
                wm_new.append(jnp.maximum(wm[h], acc))
            return tuple(wm_new)

        wmaxs = lax.fori_loop(0, _KA // 16, group_body, wmaxs)
        pltpu.sync_copy(alb, alpha_hbm.at[pl.ds(cidx * (HEADS * _KA),
                                                HEADS * _KA)])
        pltpu.sync_copy(eab, ea_hbm.at[pl.ds(cidx * (3 * _KA), 3 * _KA)])
        return wmaxs

    neg = jnp.full((16,), -jnp.inf, jnp.float32)
    wmaxs = (neg, neg, neg, neg)

    def chunk_body(i, wm):
        return process_chunk(w + i * _NW, wm)

    n_chunks = (_NCHUNKS // _NW) + jnp.where(w < _NCHUNKS % _NW, 1, 0)
    wmaxs = lax.fori_loop(0, n_chunks, chunk_body, wmaxs)
    for h in range(HEADS):
        maxb[pl.ds(h * 16, 16)] = wmaxs[h]
    pltpu.sync_copy(maxb, wmax_hbm.at[pl.ds(w * (HEADS * 16), HEADS * 16)])


def _pass_a(x_l, x_r, nuv_flat, src, dst, we_flat, att_flat):
    mesh = plsc.VectorSubcoreMesh(core_axis_name="c", subcore_axis_name="s",
                                  num_cores=_NC, num_subcores=_NS)
    f = pl.kernel(
        _pa_body,
        out_type=[
            jax.ShapeDtypeStruct((E * HEADS,), jnp.float32),
            jax.ShapeDtypeStruct((E * 3,), jnp.float32),
            jax.ShapeDtypeStruct((_NW * HEADS * 16,), jnp.float32),
        ],
        mesh=mesh,
        scratch_types=[
            pltpu.VMEM((2 * N,), jnp.float32),
            pltpu.VMEM((3 * HID,), jnp.float32),
            pltpu.VMEM((HEADS * CH,), jnp.float32),
            pltpu.VMEM((_KA,), jnp.int32),
            pltpu.VMEM((_KA,), jnp.int32),
            pltpu.VMEM((_KA, HID), jnp.float32),
            pltpu.VMEM((_KA, HID), jnp.float32),
            pltpu.VMEM((HEADS * _KA,), jnp.float32),
            pltpu.VMEM((3 * _KA,), jnp.float32),
            pltpu.VMEM((HEADS * 16,), jnp.float32),
            pltpu.SemaphoreType.DMA,
        ],
        compiler_params=pltpu.CompilerParams(needs_layout_passes=False,
                                             use_tc_tiling_on_sc=False),
    )
    return f(x_l, x_r, nuv_flat, src, dst, we_flat, att_flat)


_DENP = 40960
_DSL = _DENP // _NS


def _pb_body(alpha_hbm, dst_hbm, wmax_hbm, den_hbm,
             wmv, alb, dstb, den4, tmpv, shden, sem):
    c = lax.axis_index("c")
    s = lax.axis_index("s")
    w = s * _NC + c
    pltpu.sync_copy(wmax_hbm, wmv)

    def mxb(i, m):
        return jnp.maximum(m, wmv[pl.ds(i * 16, 16)])
    gv = lax.fori_loop(0, (_NW * HEADS * 16) // 16, mxb,
                       jnp.full((16,), -jnp.inf, jnp.float32))
    gmax = jnp.max(gv)

    def zb(i, cc):
        den4[pl.ds(i * 16, 16)] = jnp.zeros((16,), jnp.float32)
        return cc
    lax.fori_loop(0, _DENP // 16, zb, 0)

    def chunk(i, cc):
        cidx = w + i * _NW
        pltpu.sync_copy(dst_hbm.at[pl.ds(cidx * _KA, _KA)], dstb)
        pltpu.sync_copy(alpha_hbm.at[pl.ds(cidx * (HEADS * _KA),
                                           HEADS * _KA)], alb)

        def grp(g, c2):
            d4 = dstb[pl.ds(g * 16, 16)] * 4
            for h in range(HEADS):
                v = jnp.exp(alb[pl.ds(h * _KA + g * 16, 16)] - gmax)
                plsc.addupdate_scatter(den4, [d4 + h], v)
            return c2
        return lax.fori_loop(0, _KA // 16, grp, cc)

    n_chunks = (_NCHUNKS // _NW) + jnp.where(w < _NCHUNKS % _NW, 1, 0)
    lax.fori_loop(0, n_chunks, chunk, 0)

    pltpu.sync_copy(den4, shden.at[s])
    plsc.subcore_barrier()
    off = s * _DSL
    pltpu.sync_copy(shden.at[0, pl.ds(off, _DSL)], den4.at[pl.ds(0, _DSL)])

    def redt(t, cc):
        pltpu.sync_copy(shden.at[t, pl.ds(off, _DSL)], tmpv)

        def addv(i, c3):
            den4[pl.ds(i * 16, 16)] = (den4[pl.ds(i * 16, 16)]
                                       + tmpv[pl.ds(i * 16, 16)])
            return c3
        return lax.fori_loop(0, _DSL // 16, addv, cc)
    lax.fori_loop(1, _NS, redt, 0)
    pltpu.sync_copy(den4.at[pl.ds(0, _DSL)],
                    den_hbm.at[c, pl.ds(off, _DSL)])


def _pass_b(alpha_f, dst, wmax):
    mesh = plsc.VectorSubcoreMesh(core_axis_name="c", subcore_axis_name="s",
                                  num_cores=_NC, num_subcores=_NS)
    f = pl.kernel(
        _pb_body,
        out_type=[jax.ShapeDtypeStruct((_NC, _DENP), jnp.float32)],
        mesh=mesh,
        scratch_types=[
            pltpu.VMEM((_NW * HEADS * 16,), jnp.float32),
            pltpu.VMEM((HEADS * _KA,), jnp.float32),
            pltpu.VMEM((_KA,), jnp.int32),
            pltpu.VMEM((_DENP,), jnp.float32),
            pltpu.VMEM((_DSL,), jnp.float32),
            pltpu.VMEM_SHARED((_NS, _DENP), jnp.float32),
            pltpu.SemaphoreType.DMA,
        ],
        compiler_params=pltpu.CompilerParams(needs_layout_passes=False,
                                             use_tc_tiling_on_sc=False),
    )
    return f(alpha_f, dst, wmax)[0]


def _pc_body(alpha_hbm, src_hbm, dst_hbm, wmax_hbm, den_hbm,
             xq0_hbm, xq1_hbm, xq2_hbm, xq3_hbm,
             asm_hbm, a0_hbm, a1_hbm, a2_hbm, a3_hbm,
             wmv, den4, tmpv, alb1, asmb, srcb, dstb, rows, accsh, sem):
    c = lax.axis_index("c")
    s = lax.axis_index("s")
    pltpu.sync_copy(wmax_hbm, wmv)

    def mxb(i, m):
        return jnp.maximum(m, wmv[pl.ds(i * 16, 16)])
    gv = lax.fori_loop(0, (_NW * HEADS * 16) // 16, mxb,
                       jnp.full((16,), -jnp.inf, jnp.float32))
    gmax = jnp.max(gv)

    pltpu.sync_copy(den_hbm.at[0], den4)

    def dsum(i, cc):
        pltpu.sync_copy(den_hbm.at[1, pl.ds(i * _DSL, _DSL)], tmpv)

        def addv(j, c3):
            den4[pl.ds(i * _DSL + j * 16, 16)] = (
                den4[pl.ds(i * _DSL + j * 16, 16)] + tmpv[pl.ds(j * 16, 16)])
            return c3
        return lax.fori_loop(0, _DSL // 16, addv, cc)
    lax.fori_loop(0, _NS, dsum, 0)

    rows_per = N // _NS
    xqs = (xq0_hbm, xq1_hbm, xq2_hbm, xq3_hbm)
    accs = (a0_hbm, a1_hbm, a2_hbm, a3_hbm)

    def zero_acc():
        def zrow(i, cc):
            for v in range(4):
                rows[i, pl.ds(v * 16, 16)] = jnp.zeros((16,), jnp.float32)
            return cc
        lax.fori_loop(0, _KA, zrow, 0)

        def zacc(i, cc):
            pltpu.sync_copy(rows,
                            accsh.at[pl.ds(s * rows_per + i * _KA, _KA)])
            return cc
        lax.fori_loop(0, rows_per // _KA, zacc, 0)
        pltpu.sync_copy(
            rows.at[pl.ds(0, rows_per % _KA)],
            accsh.at[pl.ds(s * rows_per + (rows_per // _KA) * _KA,
                           rows_per % _KA)])

    def edge_loop(xq_hbm, h):
        def chunk(i, cc):
            cidx = s + i * _NS
            base = cidx * _KA
            pltpu.sync_copy(src_hbm.at[pl.ds(base, _KA)], srcb)
            pltpu.sync_copy(dst_hbm.at[pl.ds(base, _KA)], dstb)
            pltpu.sync_copy(
                alpha_hbm.at[pl.ds(cidx * (HEADS * _KA) + h * _KA, _KA)],
                alb1)
            pltpu.async_copy(xq_hbm.at[srcb], rows, sem).wait()

            def grp(g, c2):
                dst_v = dstb[pl.ds(g * 16, 16)]
                av = alb1[pl.ds(g * 16, 16)]
                ex = jnp.exp(av - gmax)
                dg = plsc.load_gather(den4, [dst_v * 4 + h])
                asm = ex / (dg + 1e-16)
                asmb[pl.ds(g * 16, 16)] = asm
                return c2
            lax.fori_loop(0, _KA // 16, grp, cc)
            pltpu.sync_copy(
                asmb.at[pl.ds(0, _KA)],
                asm_hbm.at[pl.ds(cidx * (HEADS * _KA) + h * _KA, _KA)])

            def scale(e, c2):
                a0 = asmb[pl.ds(e, 16)][0]
                for v in range(4):
                    rows[e, pl.ds(v * 16, 16)] = (rows[e, pl.ds(v * 16, 16)]
                                                  * a0)
                return c2
            lax.fori_loop(0, _KA, scale, cc)
            pltpu.sync_copy(rows, accsh.at[dstb], add=True)
            return cc
        n_chunks = (_NCHUNKS // _NS) + jnp.where(s < _NCHUNKS % _NS, 1, 0)
        lax.fori_loop(0, n_chunks, chunk, 0)

    for p in range(2):
        zero_acc()
        plsc.subcore_barrier()

        @pl.when(c == 0)
        def _(_p=p):
            edge_loop(xqs[2 * _p], 2 * _p)

        @pl.when(c == 1)
        def _(_p=p):
            edge_loop(xqs[2 * _p + 1], 2 * _p + 1)

        plsc.subcore_barrier()

        @pl.when(c == 0)
        def _(_p=p):
            pltpu.sync_copy(accsh.at[pl.ds(s * rows_per, rows_per)],
                            accs[2 * _p].at[pl.ds(s * rows_per, rows_per)])

        @pl.when(c == 1)
        def _(_p=p):
            pltpu.sync_copy(accsh.at[pl.ds(s * rows_per, rows_per)],
                            accs[2 * _p + 1].at[pl.ds(s * rows_per,
                                                      rows_per)])

        plsc.subcore_barrier()


def _pass_c(alpha_f, src, dst, wmax, den_p, xq0, xq1, xq2, xq3):
    mesh = plsc.VectorSubcoreMesh(core_axis_name="c", subcore_axis_name="s",
                                  num_cores=_NC, num_subcores=_NS)
    f = pl.kernel(
        _pc_body,
        out_type=[
            jax.ShapeDtypeStruct((E * HEADS,), jnp.float32),
            jax.ShapeDtypeStruct((N, CH), jnp.float32),
            jax.ShapeDtypeStruct((N, CH), jnp.float32),
            jax.ShapeDtypeStruct((N, CH), jnp.float32),
            jax.ShapeDtypeStruct((N, CH), jnp.float32),
        ],
        mesh=mesh,
        scratch_types=[
            pltpu.VMEM((_NW * HEADS * 16,), jnp.float32),
            pltpu.VMEM((_DENP,), jnp.float32),
            pltpu.VMEM((_DSL,), jnp.float32),
            pltpu.VMEM((_KA,), jnp.float32),
            pltpu.VMEM((_KA + 16,), jnp.float32),
            pltpu.VMEM((_KA,), jnp.int32),
            pltpu.VMEM((_KA,), jnp.int32),
            pltpu.VMEM((_KA, CH), jnp.float32),
            pltpu.VMEM_SHARED((N, CH), jnp.float32),
            pltpu.SemaphoreType.DMA,
        ],
        compiler_params=pltpu.CompilerParams(needs_layout_passes=False,
                                             use_tc_tiling_on_sc=False),
    )
    return f(alpha_f, src, dst, wmax, den_p, xq0, xq1, xq2, xq3)


def kernel(x, edge_index, kpts, pts_3d, W_pe1, b_pe1, W_pe2, b_pe2, W_res,
           b_res, W_l, b_l, W_r, b_r, W_e, att, b_conv, ln_g, ln_b, W_proj,
           b_proj):
    x_l, x_r, identity, norm_uv, xq0, xq1, xq2, xq3 = _front(
        x, kpts, pts_3d, W_pe1, b_pe1, W_pe2, b_pe2, W_res, b_res,
        W_l, b_l, W_r, b_r)

    src = edge_index[0]
    dst = edge_index[1]
    alpha_f, ea_f, wmax = _pass_a(x_l, x_r, norm_uv.reshape(-1), src, dst,
                                  W_e.reshape(-1), att.reshape(-1))
    nch = E // _KA
    edge_attr = ea_f.reshape(nch, 3, _KA).transpose(0, 2, 1).reshape(E, 3)

    den_p = _pass_b(alpha_f, dst, wmax)
    asm_f, a0, a1, a2, a3 = _pass_c(alpha_f, src, dst, wmax, den_p,
                                    xq0, xq1, xq2, xq3)
    alpha_sm = asm_f.reshape(nch, HEADS, _KA).transpose(0, 2, 1).reshape(
        E, HEADS)
    acc = jnp.concatenate([a0, a1, a2, a3], axis=1)

    out = _tail(acc, identity, b_conv, ln_g, ln_b, W_proj, b_proj)
    return (out, alpha_sm, edge_attr)


# speedup vs baseline: 11.0738x; 1.4264x over previous
import functools

import jax
import jax.numpy as jnp
from jax import lax
from jax.experimental import pallas as pl
from jax.experimental.pallas import tpu as pltpu
from jax.experimental.pallas import tpu_sc as plsc

N = 10000
E = 160000
D_IN = 256
D_CAT = 320
HID = 256
HEADS = 4
CH = 64

_BN = 1000


def _sigmoid(v):
    return 1.0 / (1.0 + jnp.exp(-v))


def _front_body(x_ref, kpts_ref, pts_ref, wpe1_ref, bpe1_ref, wpe2_ref,
                bpe2_ref, wres_ref, bres_ref, wl_ref, bl_ref, wr_ref, br_ref,
                xl_ref, xr_ref, ident_ref, nuv_ref, xq0_ref, xq1_ref, xq2_ref, xq3_ref):
    kpts = kpts_ref[...]
    nu = kpts[:, 0:1] * (1.0 / 1216.0)
    nv = kpts[:, 1:2] * (1.0 / 352.0)
    depth = pts_ref[:, 2:3]
    w1 = wpe1_ref[...]
    h = (nu * w1[0:1, :] + nv * w1[1:2, :] + depth * w1[2:3, :] + bpe1_ref[...])
    h = h * _sigmoid(h)
    pos = jnp.dot(h, wpe2_ref[...], preferred_element_type=jnp.float32) + bpe2_ref[...]
    x = x_ref[...]

    def lin(w_ref, b_ref):
        w = w_ref[...]
        return (jnp.dot(x, w[:D_IN, :], preferred_element_type=jnp.float32)
                + jnp.dot(pos, w[D_IN:, :], preferred_element_type=jnp.float32)
                + b_ref[...])

    xl = lin(wl_ref, bl_ref)
    xl_ref[...] = xl
    xq0_ref[...] = xl[:, 0:CH]
    xq1_ref[...] = xl[:, CH:2 * CH]
    xq2_ref[...] = xl[:, 2 * CH:3 * CH]
    xq3_ref[...] = xl[:, 3 * CH:]
    xr_ref[...] = lin(wr_ref, br_ref)
    ident_ref[...] = lin(wres_ref, bres_ref)
    nuv_ref[...] = jnp.concatenate([nu, nv], axis=1)


def _front(x, kpts, pts_3d, W_pe1, b_pe1, W_pe2, b_pe2, W_res, b_res,
           W_l, b_l, W_r, b_r):
    grid = (N // _BN,)
    bspec = lambda shp: pl.BlockSpec(shp, lambda i: (i, 0))
    fspec = lambda shp: pl.BlockSpec(shp, lambda i: (0, 0))
    row = lambda d: pl.BlockSpec((d,), lambda i: (0,))
    return pl.pallas_call(
        _front_body,
        grid=grid,
        in_specs=[
            bspec((_BN, D_IN)), bspec((_BN, 2)), bspec((_BN, 3)),
            fspec((3, 32)), row(32), fspec((32, 64)), row(64),
            fspec((D_CAT, HID)), row(HID),
            fspec((D_CAT, HID)), row(HID),
            fspec((D_CAT, HID)), row(HID),
        ],
        out_specs=[
            bspec((_BN, HID)), bspec((_BN, HID)), bspec((_BN, HID)),
            bspec((_BN, 2)), bspec((_BN, CH)), bspec((_BN, CH)),
            bspec((_BN, CH)), bspec((_BN, CH)),
        ],
        out_shape=[
            jax.ShapeDtypeStruct((N, HID), jnp.float32),
            jax.ShapeDtypeStruct((N, HID), jnp.float32),
            jax.ShapeDtypeStruct((N, HID), jnp.float32),
            jax.ShapeDtypeStruct((N, 2), jnp.float32),
            jax.ShapeDtypeStruct((N, CH), jnp.float32),
            jax.ShapeDtypeStruct((N, CH), jnp.float32),
            jax.ShapeDtypeStruct((N, CH), jnp.float32),
            jax.ShapeDtypeStruct((N, CH), jnp.float32),
        ],
    )(x, kpts, pts_3d, W_pe1, b_pe1, W_pe2, b_pe2, W_res, b_res,
      W_l, b_l, W_r, b_r)


def _tail_body(acc_ref, ident_ref, bconv_ref, lng_ref, lnb_ref, wproj_ref,
               bproj_ref, out_ref):
    o = acc_ref[...] + bconv_ref[...]
    mu = jnp.mean(o, axis=-1, keepdims=True)
    var = jnp.mean((o - mu) ** 2, axis=-1, keepdims=True)
    o = (o - mu) / jnp.sqrt(var + 1e-5) * lng_ref[...] + lnb_ref[...]
    o = o * _sigmoid(o)
    o = o + ident_ref[...]
    out_ref[...] = (jnp.dot(o, wproj_ref[...], preferred_element_type=jnp.float32)
                    + bproj_ref[...])


def _tail(acc, ident, b_conv, ln_g, ln_b, W_proj, b_proj):
    grid = (N // _BN,)
    bspec = lambda shp: pl.BlockSpec(shp, lambda i: (i, 0))
    fspec = lambda shp: pl.BlockSpec(shp, lambda i: (0, 0))
    row = lambda d: pl.BlockSpec((d,), lambda i: (0,))
    return pl.pallas_call(
        _tail_body,
        grid=grid,
        in_specs=[
            bspec((_BN, HID)), bspec((_BN, HID)), row(HID),
            row(HID), row(HID), fspec((HID, 256)), row(256),
        ],
        out_specs=pl.BlockSpec((_BN, 256), lambda i: (i, 0)),
        out_shape=jax.ShapeDtypeStruct((N, 256), jnp.float32),
    )(acc, ident, b_conv, ln_g, ln_b, W_proj, b_proj)


_NC = 2
_NS = 16
_NW = _NC * _NS
_EPW = E // _NW
_KA = 128
_NCHUNKS = E // _KA


def _nsqrt(d2):
    i = plsc.bitcast(d2, jnp.int32)
    i = jnp.int32(0x5F3759DF) - lax.shift_right_logical(i, jnp.int32(1))
    y = plsc.bitcast(i, jnp.float32)
    for _ in range(3):
        y = y * (1.5 - 0.5 * d2 * y * y)
    return jnp.where(d2 > 0.0, d2 * y, 0.0)


def _pa_body(xl_hbm, xr_hbm, nuv_hbm, src_hbm, dst_hbm, we_hbm, att_hbm,
             alpha_hbm, ea_hbm, wmax_hbm,
             nuv_v, we_v, att_v, srcb, dstb, xlb, xrb, xf, alb, eab, maxb,
             sem):
    w = lax.axis_index("s") * _NC + lax.axis_index("c")
    pltpu.sync_copy(nuv_hbm, nuv_v)
    pltpu.sync_copy(we_hbm, we_v)
    pltpu.sync_copy(att_hbm, att_v)
    iota16 = lax.iota(jnp.int32, 16)

    def process_chunk(cidx, wmaxs):
        base = cidx * _KA
        pltpu.sync_copy(src_hbm.at[pl.ds(base, _KA)], srcb)
        pltpu.sync_copy(dst_hbm.at[pl.ds(base, _KA)], dstb)
        pltpu.async_copy(xl_hbm.at[srcb], xlb, sem).wait()
        pltpu.async_copy(xr_hbm.at[dstb], xrb, sem).wait()

        def repack(e, cc):
            ibase = e * 257 + iota16
            for t in range(HID // 16):
                v = xlb[e, pl.ds(t * 16, 16)] + xrb[e, pl.ds(t * 16, 16)]
                plsc.store_scatter(xf, [ibase + t * 16], v)
            return cc
        lax.fori_loop(0, _KA, repack, 0)

        def group_body(g, wm):
            sl = pl.ds(g * 16, 16)
            evec = (iota16 + g * 16) * 257
            src2 = srcb[sl] * 2
            dst2 = dstb[sl] * 2
            u_s = plsc.load_gather(nuv_v, [src2])
            v_s = plsc.load_gather(nuv_v, [src2 + 1])
            u_d = plsc.load_gather(nuv_v, [dst2])
            v_d = plsc.load_gather(nuv_v, [dst2 + 1])
            ru = u_d - u_s
            rv = v_d - v_s
            dist = _nsqrt(ru * ru + rv * rv)
            eab[pl.ds(g * 16, 16)] = ru
            eab[pl.ds(_KA + g * 16, 16)] = rv
            eab[pl.ds(2 * _KA + g * 16, 16)] = dist
            wm_new = []
            for h in range(HEADS):
                def cbody(cb, acc, _h=h):
                    cbase = _h * CH + cb * 16
                    w0v = we_v[pl.ds(cbase, 16)]
                    w1v = we_v[pl.ds(HID + cbase, 16)]
                    w2v = we_v[pl.ds(2 * HID + cbase, 16)]
                    atv = att_v[pl.ds(_h * CH + cb * 16, 16)]
                    for j in range(16):
                        ms = plsc.load_gather(xf, [evec + (cbase + j)])
                        ef = ru * w0v[j] + rv * w1v[j] + dist * w2v[j]
                        m = ms + ef
                        lk = jnp.maximum(m, 0.2 * m)
                        acc = acc + atv[j] * lk
                    return acc
                acc = lax.fori_loop(0, CH // 16, cbody,
                                    jnp.zeros((16,), jnp.float32))
                alb[pl.ds(h * _KA + g * 16, 16)] = acc
                wm_new.append(jnp.maximum(wm[h], acc))
            return tuple(wm_new)

        wmaxs = lax.fori_loop(0, _KA // 16, group_body, wmaxs)
        pltpu.sync_copy(alb, alpha_hbm.at[pl.ds(cidx * (HEADS * _KA),
                                                HEADS * _KA)])
        pltpu.sync_copy(eab, ea_hbm.at[pl.ds(cidx * (3 * _KA), 3 * _KA)])
        return wmaxs

    neg = jnp.full((16,), -jnp.inf, jnp.float32)
    wmaxs = (neg, neg, neg, neg)

    def chunk_body(i, wm):
        return process_chunk(w + i * _NW, wm)

    n_chunks = (_NCHUNKS // _NW) + jnp.where(w < _NCHUNKS % _NW, 1, 0)
    wmaxs = lax.fori_loop(0, n_chunks, chunk_body, wmaxs)
    for h in range(HEADS):
        maxb[pl.ds(h * 16, 16)] = wmaxs[h]
    pltpu.sync_copy(maxb, wmax_hbm.at[pl.ds(w * (HEADS * 16), HEADS * 16)])


def _pass_a(x_l, x_r, nuv_flat, src, dst, we_flat, att_flat):
    mesh = plsc.VectorSubcoreMesh(core_axis_name="c", subcore_axis_name="s",
                                  num_cores=_NC, num_subcores=_NS)
    f = pl.kernel(
        _pa_body,
        out_type=[
            jax.ShapeDtypeStruct((E * HEADS,), jnp.float32),
            jax.ShapeDtypeStruct((E * 3,), jnp.float32),
            jax.ShapeDtypeStruct((_NW * HEADS * 16,), jnp.float32),
        ],
        mesh=mesh,
        scratch_types=[
            pltpu.VMEM((2 * N,), jnp.float32),
            pltpu.VMEM((3 * HID,), jnp.float32),
            pltpu.VMEM((HEADS * CH,), jnp.float32),
            pltpu.VMEM((_KA,), jnp.int32),
            pltpu.VMEM((_KA,), jnp.int32),
            pltpu.VMEM((_KA, HID), jnp.float32),
            pltpu.VMEM((_KA, HID), jnp.float32),
            pltpu.VMEM((_KA * 257,), jnp.float32),
            pltpu.VMEM((HEADS * _KA,), jnp.float32),
            pltpu.VMEM((3 * _KA,), jnp.float32),
            pltpu.VMEM((HEADS * 16,), jnp.float32),
            pltpu.SemaphoreType.DMA,
        ],
        compiler_params=pltpu.CompilerParams(needs_layout_passes=False,
                                             use_tc_tiling_on_sc=False),
    )
    return f(x_l, x_r, nuv_flat, src, dst, we_flat, att_flat)


_DENP = 40960
_DSL = _DENP // _NS


def _pb_body(alpha_hbm, dst_hbm, wmax_hbm, den_hbm,
             wmv, alb, dstb, den4, tmpv, shden, sem):
    c = lax.axis_index("c")
    s = lax.axis_index("s")
    w = s * _NC + c
    pltpu.sync_copy(wmax_hbm, wmv)

    def mxb(i, m):
        return jnp.maximum(m, wmv[pl.ds(i * 16, 16)])
    gv = lax.fori_loop(0, (_NW * HEADS * 16) // 16, mxb,
                       jnp.full((16,), -jnp.inf, jnp.float32))
    gmax = jnp.max(gv)

    def zb(i, cc):
        den4[pl.ds(i * 16, 16)] = jnp.zeros((16,), jnp.float32)
        return cc
    lax.fori_loop(0, _DENP // 16, zb, 0)

    def chunk(i, cc):
        cidx = w + i * _NW
        pltpu.sync_copy(dst_hbm.at[pl.ds(cidx * _KA, _KA)], dstb)
        pltpu.sync_copy(alpha_hbm.at[pl.ds(cidx * (HEADS * _KA),
                                           HEADS * _KA)], alb)

        def grp(g, c2):
            d4 = dstb[pl.ds(g * 16, 16)] * 4
            for h in range(HEADS):
                v = jnp.exp(alb[pl.ds(h * _KA + g * 16, 16)] - gmax)
                plsc.addupdate_scatter(den4, [d4 + h], v)
            return c2
        return lax.fori_loop(0, _KA // 16, grp, cc)

    n_chunks = (_NCHUNKS // _NW) + jnp.where(w < _NCHUNKS % _NW, 1, 0)
    lax.fori_loop(0, n_chunks, chunk, 0)

    pltpu.sync_copy(den4, shden.at[s])
    plsc.subcore_barrier()
    off = s * _DSL
    pltpu.sync_copy(shden.at[0, pl.ds(off, _DSL)], den4.at[pl.ds(0, _DSL)])

    def redt(t, cc):
        pltpu.sync_copy(shden.at[t, pl.ds(off, _DSL)], tmpv)

        def addv(i, c3):
            den4[pl.ds(i * 16, 16)] = (den4[pl.ds(i * 16, 16)]
                                       + tmpv[pl.ds(i * 16, 16)])
            return c3
        return lax.fori_loop(0, _DSL // 16, addv, cc)
    lax.fori_loop(1, _NS, redt, 0)
    pltpu.sync_copy(den4.at[pl.ds(0, _DSL)],
                    den_hbm.at[c, pl.ds(off, _DSL)])


def _pass_b(alpha_f, dst, wmax):
    mesh = plsc.VectorSubcoreMesh(core_axis_name="c", subcore_axis_name="s",
                                  num_cores=_NC, num_subcores=_NS)
    f = pl.kernel(
        _pb_body,
        out_type=[jax.ShapeDtypeStruct((_NC, _DENP), jnp.float32)],
        mesh=mesh,
        scratch_types=[
            pltpu.VMEM((_NW * HEADS * 16,), jnp.float32),
            pltpu.VMEM((HEADS * _KA,), jnp.float32),
            pltpu.VMEM((_KA,), jnp.int32),
            pltpu.VMEM((_DENP,), jnp.float32),
            pltpu.VMEM((_DSL,), jnp.float32),
            pltpu.VMEM_SHARED((_NS, _DENP), jnp.float32),
            pltpu.SemaphoreType.DMA,
        ],
        compiler_params=pltpu.CompilerParams(needs_layout_passes=False,
                                             use_tc_tiling_on_sc=False),
    )
    return f(alpha_f, dst, wmax)[0]


def _pc_body(alpha_hbm, src_hbm, dst_hbm, wmax_hbm, den_hbm,
             xq0_hbm, xq1_hbm, xq2_hbm, xq3_hbm,
             asm_hbm, a0_hbm, a1_hbm, a2_hbm, a3_hbm,
             wmv, den4, tmpv, alb1, asmb, srcb, dstb, rows, accsh, sem):
    c = lax.axis_index("c")
    s = lax.axis_index("s")
    pltpu.sync_copy(wmax_hbm, wmv)

    def mxb(i, m):
        return jnp.maximum(m, wmv[pl.ds(i * 16, 16)])
    gv = lax.fori_loop(0, (_NW * HEADS * 16) // 16, mxb,
                       jnp.full((16,), -jnp.inf, jnp.float32))
    gmax = jnp.max(gv)

    pltpu.sync_copy(den_hbm.at[0], den4)

    def dsum(i, cc):
        pltpu.sync_copy(den_hbm.at[1, pl.ds(i * _DSL, _DSL)], tmpv)

        def addv(j, c3):
            den4[pl.ds(i * _DSL + j * 16, 16)] = (
                den4[pl.ds(i * _DSL + j * 16, 16)] + tmpv[pl.ds(j * 16, 16)])
            return c3
        return lax.fori_loop(0, _DSL // 16, addv, cc)
    lax.fori_loop(0, _NS, dsum, 0)

    rows_per = N // _NS
    xqs = (xq0_hbm, xq1_hbm, xq2_hbm, xq3_hbm)
    accs = (a0_hbm, a1_hbm, a2_hbm, a3_hbm)

    def zero_acc():
        def zrow(i, cc):
            for v in range(4):
                rows[i, pl.ds(v * 16, 16)] = jnp.zeros((16,), jnp.float32)
            return cc
        lax.fori_loop(0, _KA, zrow, 0)

        def zacc(i, cc):
            pltpu.sync_copy(rows,
                            accsh.at[pl.ds(s * rows_per + i * _KA, _KA)])
            return cc
        lax.fori_loop(0, rows_per // _KA, zacc, 0)
        pltpu.sync_copy(
            rows.at[pl.ds(0, rows_per % _KA)],
            accsh.at[pl.ds(s * rows_per + (rows_per // _KA) * _KA,
                           rows_per % _KA)])

    def edge_loop(xq_hbm, h):
        def chunk(i, cc):
            cidx = s + i * _NS
            base = cidx * _KA
            pltpu.sync_copy(src_hbm.at[pl.ds(base, _KA)], srcb)
            pltpu.sync_copy(dst_hbm.at[pl.ds(base, _KA)], dstb)
            pltpu.sync_copy(
                alpha_hbm.at[pl.ds(cidx * (HEADS * _KA) + h * _KA, _KA)],
                alb1)
            pltpu.async_copy(xq_hbm.at[srcb], rows, sem).wait()

            def grp(g, c2):
                dst_v = dstb[pl.ds(g * 16, 16)]
                av = alb1[pl.ds(g * 16, 16)]
                ex = jnp.exp(av - gmax)
                dg = plsc.load_gather(den4, [dst_v * 4 + h])
                asm = ex / (dg + 1e-16)
                asmb[pl.ds(g * 16, 16)] = asm
                return c2
            lax.fori_loop(0, _KA // 16, grp, cc)
            pltpu.sync_copy(
                asmb.at[pl.ds(0, _KA)],
                asm_hbm.at[pl.ds(cidx * (HEADS * _KA) + h * _KA, _KA)])

            def scale(e, c2):
                a0 = asmb[pl.ds(e, 16)][0]
                for v in range(4):
                    rows[e, pl.ds(v * 16, 16)] = (rows[e, pl.ds(v * 16, 16)]
                                                  * a0)
                return c2
            lax.fori_loop(0, _KA, scale, cc)
            pltpu.sync_copy(rows, accsh.at[dstb], add=True)
            return cc
        n_chunks = (_NCHUNKS // _NS) + jnp.where(s < _NCHUNKS % _NS, 1, 0)
        lax.fori_loop(0, n_chunks, chunk, 0)

    for p in range(2):
        zero_acc()
        plsc.subcore_barrier()

        @pl.when(c == 0)
        def _(_p=p):
            edge_loop(xqs[2 * _p], 2 * _p)

        @pl.when(c == 1)
        def _(_p=p):
            edge_loop(xqs[2 * _p + 1], 2 * _p + 1)

        plsc.subcore_barrier()

        @pl.when(c == 0)
        def _(_p=p):
            pltpu.sync_copy(accsh.at[pl.ds(s * rows_per, rows_per)],
                            accs[2 * _p].at[pl.ds(s * rows_per, rows_per)])

        @pl.when(c == 1)
        def _(_p=p):
            pltpu.sync_copy(accsh.at[pl.ds(s * rows_per, rows_per)],
                            accs[2 * _p + 1].at[pl.ds(s * rows_per,
                                                      rows_per)])

        plsc.subcore_barrier()


def _pass_c(alpha_f, src, dst, wmax, den_p, xq0, xq1, xq2, xq3):
    mesh = plsc.VectorSubcoreMesh(core_axis_name="c", subcore_axis_name="s",
                                  num_cores=_NC, num_subcores=_NS)
    f = pl.kernel(
        _pc_body,
        out_type=[
            jax.ShapeDtypeStruct((E * HEADS,), jnp.float32),
            jax.ShapeDtypeStruct((N, CH), jnp.float32),
            jax.ShapeDtypeStruct((N, CH), jnp.float32),
            jax.ShapeDtypeStruct((N, CH), jnp.float32),
            jax.ShapeDtypeStruct((N, CH), jnp.float32),
        ],
        mesh=mesh,
        scratch_types=[
            pltpu.VMEM((_NW * HEADS * 16,), jnp.float32),
            pltpu.VMEM((_DENP,), jnp.float32),
            pltpu.VMEM((_DSL,), jnp.float32),
            pltpu.VMEM((_KA,), jnp.float32),
            pltpu.VMEM((_KA + 16,), jnp.float32),
            pltpu.VMEM((_KA,), jnp.int32),
            pltpu.VMEM((_KA,), jnp.int32),
            pltpu.VMEM((_KA, CH), jnp.float32),
            pltpu.VMEM_SHARED((N, CH), jnp.float32),
            pltpu.SemaphoreType.DMA,
        ],
        compiler_params=pltpu.CompilerParams(needs_layout_passes=False,
                                             use_tc_tiling_on_sc=False),
    )
    return f(alpha_f, src, dst, wmax, den_p, xq0, xq1, xq2, xq3)


def kernel(x, edge_index, kpts, pts_3d, W_pe1, b_pe1, W_pe2, b_pe2, W_res,
           b_res, W_l, b_l, W_r, b_r, W_e, att, b_conv, ln_g, ln_b, W_proj,
           b_proj):
    x_l, x_r, identity, norm_uv, xq0, xq1, xq2, xq3 = _front(
        x, kpts, pts_3d, W_pe1, b_pe1, W_pe2, b_pe2, W_res, b_res,
        W_l, b_l, W_r, b_r)

    src = edge_index[0]
    dst = edge_index[1]
    alpha_f, ea_f, wmax = _pass_a(x_l, x_r, norm_uv.reshape(-1), src, dst,
                                  W_e.reshape(-1), att.reshape(-1))
    nch = E // _KA
    edge_attr = ea_f.reshape(nch, 3, _KA).transpose(0, 2, 1).reshape(E, 3)

    den_p = _pass_b(alpha_f, dst, wmax)
    asm_f, a0, a1, a2, a3 = _pass_c(alpha_f, src, dst, wmax, den_p,
                                    xq0, xq1, xq2, xq3)
    alpha_sm = asm_f.reshape(nch, HEADS, _KA).transpose(0, 2, 1).reshape(
        E, HEADS)
    acc = jnp.concatenate([a0, a1, a2, a3], axis=1)

    out = _tail(acc, identity, b_conv, ln_g, ln_b, W_proj, b_proj)
    return (out, alpha_sm, edge_attr)


# speedup vs baseline: 12.1800x; 1.0999x over previous
import functools

import jax
import jax.numpy as jnp
from jax import lax
from jax.experimental import pallas as pl
from jax.experimental.pallas import tpu as pltpu
from jax.experimental.pallas import tpu_sc as plsc

N = 10000
E = 160000
D_IN = 256
D_CAT = 320
HID = 256
HEADS = 4
CH = 64

_BN = 1000


def _sigmoid(v):
    return 1.0 / (1.0 + jnp.exp(-v))


def _front_body(x_ref, kpts_ref, pts_ref, wpe1_ref, bpe1_ref, wpe2_ref,
                bpe2_ref, wres_ref, bres_ref, wl_ref, bl_ref, wr_ref, br_ref,
                xl_ref, xr_ref, ident_ref, nuv_ref, xq0_ref, xq1_ref, xq2_ref, xq3_ref):
    kpts = kpts_ref[...]
    nu = kpts[:, 0:1] * (1.0 / 1216.0)
    nv = kpts[:, 1:2] * (1.0 / 352.0)
    depth = pts_ref[:, 2:3]
    w1 = wpe1_ref[...]
    h = (nu * w1[0:1, :] + nv * w1[1:2, :] + depth * w1[2:3, :] + bpe1_ref[...])
    h = h * _sigmoid(h)
    pos = jnp.dot(h, wpe2_ref[...], preferred_element_type=jnp.float32) + bpe2_ref[...]
    x = x_ref[...]

    def lin(w_ref, b_ref):
        w = w_ref[...]
        return (jnp.dot(x, w[:D_IN, :], preferred_element_type=jnp.float32)
                + jnp.dot(pos, w[D_IN:, :], preferred_element_type=jnp.float32)
                + b_ref[...])

    xl = lin(wl_ref, bl_ref)
    xl_ref[...] = xl
    xq0_ref[...] = xl[:, 0:CH]
    xq1_ref[...] = xl[:, CH:2 * CH]
    xq2_ref[...] = xl[:, 2 * CH:3 * CH]
    xq3_ref[...] = xl[:, 3 * CH:]
    xr_ref[...] = lin(wr_ref, br_ref)
    ident_ref[...] = lin(wres_ref, bres_ref)
    nuv_ref[...] = jnp.concatenate([nu, nv], axis=1)


def _front(x, kpts, pts_3d, W_pe1, b_pe1, W_pe2, b_pe2, W_res, b_res,
           W_l, b_l, W_r, b_r):
    grid = (N // _BN,)
    bspec = lambda shp: pl.BlockSpec(shp, lambda i: (i, 0))
    fspec = lambda shp: pl.BlockSpec(shp, lambda i: (0, 0))
    row = lambda d: pl.BlockSpec((d,), lambda i: (0,))
    return pl.pallas_call(
        _front_body,
        grid=grid,
        in_specs=[
            bspec((_BN, D_IN)), bspec((_BN, 2)), bspec((_BN, 3)),
            fspec((3, 32)), row(32), fspec((32, 64)), row(64),
            fspec((D_CAT, HID)), row(HID),
            fspec((D_CAT, HID)), row(HID),
            fspec((D_CAT, HID)), row(HID),
        ],
        out_specs=[
            bspec((_BN, HID)), bspec((_BN, HID)), bspec((_BN, HID)),
            bspec((_BN, 2)), bspec((_BN, CH)), bspec((_BN, CH)),
            bspec((_BN, CH)), bspec((_BN, CH)),
        ],
        out_shape=[
            jax.ShapeDtypeStruct((N, HID), jnp.float32),
            jax.ShapeDtypeStruct((N, HID), jnp.float32),
            jax.ShapeDtypeStruct((N, HID), jnp.float32),
            jax.ShapeDtypeStruct((N, 2), jnp.float32),
            jax.ShapeDtypeStruct((N, CH), jnp.float32),
            jax.ShapeDtypeStruct((N, CH), jnp.float32),
            jax.ShapeDtypeStruct((N, CH), jnp.float32),
            jax.ShapeDtypeStruct((N, CH), jnp.float32),
        ],
    )(x, kpts, pts_3d, W_pe1, b_pe1, W_pe2, b_pe2, W_res, b_res,
      W_l, b_l, W_r, b_r)


def _tail_body(acc_ref, ident_ref, bconv_ref, lng_ref, lnb_ref, wproj_ref,
               bproj_ref, out_ref):
    o = acc_ref[...] + bconv_ref[...]
    mu = jnp.mean(o, axis=-1, keepdims=True)
    var = jnp.mean((o - mu) ** 2, axis=-1, keepdims=True)
    o = (o - mu) / jnp.sqrt(var + 1e-5) * lng_ref[...] + lnb_ref[...]
    o = o * _sigmoid(o)
    o = o + ident_ref[...]
    out_ref[...] = (jnp.dot(o, wproj_ref[...], preferred_element_type=jnp.float32)
                    + bproj_ref[...])


def _tail(acc, ident, b_conv, ln_g, ln_b, W_proj, b_proj):
    grid = (N // _BN,)
    bspec = lambda shp: pl.BlockSpec(shp, lambda i: (i, 0))
    fspec = lambda shp: pl.BlockSpec(shp, lambda i: (0, 0))
    row = lambda d: pl.BlockSpec((d,), lambda i: (0,))
    return pl.pallas_call(
        _tail_body,
        grid=grid,
        in_specs=[
            bspec((_BN, HID)), bspec((_BN, HID)), row(HID),
            row(HID), row(HID), fspec((HID, 256)), row(256),
        ],
        out_specs=pl.BlockSpec((_BN, 256), lambda i: (i, 0)),
        out_shape=jax.ShapeDtypeStruct((N, 256), jnp.float32),
    )(acc, ident, b_conv, ln_g, ln_b, W_proj, b_proj)


_NC = 2
_NS = 16
_NW = _NC * _NS
_EPW = E // _NW
_KA = 128
_NCHUNKS = E // _KA


def _nsqrt(d2):
    i = plsc.bitcast(d2, jnp.int32)
    i = jnp.int32(0x5F3759DF) - lax.shift_right_logical(i, jnp.int32(1))
    y = plsc.bitcast(i, jnp.float32)
    for _ in range(3):
        y = y * (1.5 - 0.5 * d2 * y * y)
    return jnp.where(d2 > 0.0, d2 * y, 0.0)


def _pa_body(xl_hbm, xr_hbm, nuv_hbm, src_hbm, dst_hbm, we_hbm, att_hbm,
             alpha_hbm, ea_hbm, wmax_hbm,
             nuv_v, we_v, att_v, srcb, dstb, xlb, xrb, xf, alb, eab, maxb,
             sem, sem2):
    w = lax.axis_index("s") * _NC + lax.axis_index("c")
    pltpu.sync_copy(nuv_hbm, nuv_v)
    pltpu.sync_copy(we_hbm, we_v)
    pltpu.sync_copy(att_hbm, att_v)
    iota16 = lax.iota(jnp.int32, 16)

    def process_chunk(cidx, wmaxs):
        base = cidx * _KA
        d1 = pltpu.async_copy(src_hbm.at[pl.ds(base, _KA)], srcb, sem)
        d2 = pltpu.async_copy(dst_hbm.at[pl.ds(base, _KA)], dstb, sem2)
        d1.wait()
        d2.wait()
        d1 = pltpu.async_copy(xl_hbm.at[srcb], xlb, sem)
        d2 = pltpu.async_copy(xr_hbm.at[dstb], xrb, sem2)
        d1.wait()
        d2.wait()

        def repack(e, cc):
            ibase = e * 257 + iota16
            for t in range(HID // 16):
                v = xlb[e, pl.ds(t * 16, 16)] + xrb[e, pl.ds(t * 16, 16)]
                plsc.store_scatter(xf, [ibase + t * 16], v)
            return cc
        lax.fori_loop(0, _KA, repack, 0)

        def group_body(g, wm):
            sl = pl.ds(g * 16, 16)
            evec = (iota16 + g * 16) * 257
            src2 = srcb[sl] * 2
            dst2 = dstb[sl] * 2
            u_s = plsc.load_gather(nuv_v, [src2])
            v_s = plsc.load_gather(nuv_v, [src2 + 1])
            u_d = plsc.load_gather(nuv_v, [dst2])
            v_d = plsc.load_gather(nuv_v, [dst2 + 1])
            ru = u_d - u_s
            rv = v_d - v_s
            dist = _nsqrt(ru * ru + rv * rv)
            eab[pl.ds(g * 16, 16)] = ru
            eab[pl.ds(_KA + g * 16, 16)] = rv
            eab[pl.ds(2 * _KA + g * 16, 16)] = dist
            wm_new = []
            for h in range(HEADS):
                def cbody(cb, acc, _h=h):
                    cbase = _h * CH + cb * 16
                    w0v = we_v[pl.ds(cbase, 16)]
                    w1v = we_v[pl.ds(HID + cbase, 16)]
                    w2v = we_v[pl.ds(2 * HID + cbase, 16)]
                    atv = att_v[pl.ds(_h * CH + cb * 16, 16)]
                    for j in range(16):
                        ms = plsc.load_gather(xf, [evec + (cbase + j)])
                        ef = ru * w0v[j] + rv * w1v[j] + dist * w2v[j]
                        m = ms + ef
                        lk = jnp.maximum(m, 0.2 * m)
                        acc = acc + atv[j] * lk
                    return acc
                acc = lax.fori_loop(0, CH // 16, cbody,
                                    jnp.zeros((16,), jnp.float32))
                alb[pl.ds(h * _KA + g * 16, 16)] = acc
                wm_new.append(jnp.maximum(wm[h], acc))
            return tuple(wm_new)

        wmaxs = lax.fori_loop(0, _KA // 16, group_body, wmaxs)
        pltpu.sync_copy(alb, alpha_hbm.at[pl.ds(cidx * (HEADS * _KA),
                                                HEADS * _KA)])
        pltpu.sync_copy(eab, ea_hbm.at[pl.ds(cidx * (3 * _KA), 3 * _KA)])
        return wmaxs

    neg = jnp.full((16,), -jnp.inf, jnp.float32)
    wmaxs = (neg, neg, neg, neg)

    def chunk_body(i, wm):
        return process_chunk(w + i * _NW, wm)

    n_chunks = (_NCHUNKS // _NW) + jnp.where(w < _NCHUNKS % _NW, 1, 0)
    wmaxs = lax.fori_loop(0, n_chunks, chunk_body, wmaxs)
    for h in range(HEADS):
        maxb[pl.ds(h * 16, 16)] = wmaxs[h]
    pltpu.sync_copy(maxb, wmax_hbm.at[pl.ds(w * (HEADS * 16), HEADS * 16)])


def _pass_a(x_l, x_r, nuv_flat, src, dst, we_flat, att_flat):
    mesh = plsc.VectorSubcoreMesh(core_axis_name="c", subcore_axis_name="s",
                                  num_cores=_NC, num_subcores=_NS)
    f = pl.kernel(
        _pa_body,
        out_type=[
            jax.ShapeDtypeStruct((E * HEADS,), jnp.float32),
            jax.ShapeDtypeStruct((E * 3,), jnp.float32),
            jax.ShapeDtypeStruct((_NW * HEADS * 16,), jnp.float32),
        ],
        mesh=mesh,
        scratch_types=[
            pltpu.VMEM((2 * N,), jnp.float32),
            pltpu.VMEM((3 * HID,), jnp.float32),
            pltpu.VMEM((HEADS * CH,), jnp.float32),
            pltpu.VMEM((_KA,), jnp.int32),
            pltpu.VMEM((_KA,), jnp.int32),
            pltpu.VMEM((_KA, HID), jnp.float32),
            pltpu.VMEM((_KA, HID), jnp.float32),
            pltpu.VMEM((_KA * 257,), jnp.float32),
            pltpu.VMEM((HEADS * _KA,), jnp.float32),
            pltpu.VMEM((3 * _KA,), jnp.float32),
            pltpu.VMEM((HEADS * 16,), jnp.float32),
            pltpu.SemaphoreType.DMA,
            pltpu.SemaphoreType.DMA,
        ],
        compiler_params=pltpu.CompilerParams(needs_layout_passes=False,
                                             use_tc_tiling_on_sc=False),
    )
    return f(x_l, x_r, nuv_flat, src, dst, we_flat, att_flat)


_DENP = 40960
_DSL = _DENP // _NS


def _pb_body(alpha_hbm, dst_hbm, wmax_hbm, den_hbm,
             wmv, alb, dstb, den4, tmpv, shden, sem):
    c = lax.axis_index("c")
    s = lax.axis_index("s")
    w = s * _NC + c
    pltpu.sync_copy(wmax_hbm, wmv)

    def mxb(i, m):
        return jnp.maximum(m, wmv[pl.ds(i * 16, 16)])
    gv = lax.fori_loop(0, (_NW * HEADS * 16) // 16, mxb,
                       jnp.full((16,), -jnp.inf, jnp.float32))
    gmax = jnp.max(gv)

    def zb(i, cc):
        den4[pl.ds(i * 16, 16)] = jnp.zeros((16,), jnp.float32)
        return cc
    lax.fori_loop(0, _DENP // 16, zb, 0)

    def chunk(i, cc):
        cidx = w + i * _NW
        pltpu.sync_copy(dst_hbm.at[pl.ds(cidx * _KA, _KA)], dstb)
        pltpu.sync_copy(alpha_hbm.at[pl.ds(cidx * (HEADS * _KA),
                                           HEADS * _KA)], alb)

        def grp(g, c2):
            d4 = dstb[pl.ds(g * 16, 16)] * 4
            for h in range(HEADS):
                v = jnp.exp(alb[pl.ds(h * _KA + g * 16, 16)] - gmax)
                plsc.addupdate_scatter(den4, [d4 + h], v)
            return c2
        return lax.fori_loop(0, _KA // 16, grp, cc)

    n_chunks = (_NCHUNKS // _NW) + jnp.where(w < _NCHUNKS % _NW, 1, 0)
    lax.fori_loop(0, n_chunks, chunk, 0)

    pltpu.sync_copy(den4, shden.at[s])
    plsc.subcore_barrier()
    off = s * _DSL
    pltpu.sync_copy(shden.at[0, pl.ds(off, _DSL)], den4.at[pl.ds(0, _DSL)])

    def redt(t, cc):
        pltpu.sync_copy(shden.at[t, pl.ds(off, _DSL)], tmpv)

        def addv(i, c3):
            den4[pl.ds(i * 16, 16)] = (den4[pl.ds(i * 16, 16)]
                                       + tmpv[pl.ds(i * 16, 16)])
            return c3
        return lax.fori_loop(0, _DSL // 16, addv, cc)
    lax.fori_loop(1, _NS, redt, 0)
    pltpu.sync_copy(den4.at[pl.ds(0, _DSL)],
                    den_hbm.at[c, pl.ds(off, _DSL)])


def _pass_b(alpha_f, dst, wmax):
    mesh = plsc.VectorSubcoreMesh(core_axis_name="c", subcore_axis_name="s",
                                  num_cores=_NC, num_subcores=_NS)
    f = pl.kernel(
        _pb_body,
        out_type=[jax.ShapeDtypeStruct((_NC, _DENP), jnp.float32)],
        mesh=mesh,
        scratch_types=[
            pltpu.VMEM((_NW * HEADS * 16,), jnp.float32),
            pltpu.VMEM((HEADS * _KA,), jnp.float32),
            pltpu.VMEM((_KA,), jnp.int32),
            pltpu.VMEM((_DENP,), jnp.float32),
            pltpu.VMEM((_DSL,), jnp.float32),
            pltpu.VMEM_SHARED((_NS, _DENP), jnp.float32),
            pltpu.SemaphoreType.DMA,
        ],
        compiler_params=pltpu.CompilerParams(needs_layout_passes=False,
                                             use_tc_tiling_on_sc=False),
    )
    return f(alpha_f, dst, wmax)[0]


def _pc_body(alpha_hbm, src_hbm, dst_hbm, wmax_hbm, den_hbm,
             xq0_hbm, xq1_hbm, xq2_hbm, xq3_hbm,
             asm_hbm, a0_hbm, a1_hbm, a2_hbm, a3_hbm,
             wmv, den4, tmpv, alb1, asmb, srcb, dstb, rows, accsh, sem, sem2,
             sem3):
    c = lax.axis_index("c")
    s = lax.axis_index("s")
    pltpu.sync_copy(wmax_hbm, wmv)

    def mxb(i, m):
        return jnp.maximum(m, wmv[pl.ds(i * 16, 16)])
    gv = lax.fori_loop(0, (_NW * HEADS * 16) // 16, mxb,
                       jnp.full((16,), -jnp.inf, jnp.float32))
    gmax = jnp.max(gv)

    pltpu.sync_copy(den_hbm.at[0], den4)

    def dsum(i, cc):
        pltpu.sync_copy(den_hbm.at[1, pl.ds(i * _DSL, _DSL)], tmpv)

        def addv(j, c3):
            den4[pl.ds(i * _DSL + j * 16, 16)] = (
                den4[pl.ds(i * _DSL + j * 16, 16)] + tmpv[pl.ds(j * 16, 16)])
            return c3
        return lax.fori_loop(0, _DSL // 16, addv, cc)
    lax.fori_loop(0, _NS, dsum, 0)

    rows_per = N // _NS
    xqs = (xq0_hbm, xq1_hbm, xq2_hbm, xq3_hbm)
    accs = (a0_hbm, a1_hbm, a2_hbm, a3_hbm)

    def zero_acc():
        def zrow(i, cc):
            for v in range(4):
                rows[i, pl.ds(v * 16, 16)] = jnp.zeros((16,), jnp.float32)
            return cc
        lax.fori_loop(0, _KA, zrow, 0)

        def zacc(i, cc):
            pltpu.sync_copy(rows,
                            accsh.at[pl.ds(s * rows_per + i * _KA, _KA)])
            return cc
        lax.fori_loop(0, rows_per // _KA, zacc, 0)
        pltpu.sync_copy(
            rows.at[pl.ds(0, rows_per % _KA)],
            accsh.at[pl.ds(s * rows_per + (rows_per // _KA) * _KA,
                           rows_per % _KA)])

    def edge_loop(xq_hbm, h):
        def chunk(i, cc):
            cidx = s + i * _NS
            base = cidx * _KA
            d1 = pltpu.async_copy(src_hbm.at[pl.ds(base, _KA)], srcb, sem)
            d2 = pltpu.async_copy(dst_hbm.at[pl.ds(base, _KA)], dstb, sem2)
            d3 = pltpu.async_copy(
                alpha_hbm.at[pl.ds(cidx * (HEADS * _KA) + h * _KA, _KA)],
                alb1, sem3)
            d1.wait()
            d2.wait()
            d3.wait()
            pltpu.async_copy(xq_hbm.at[srcb], rows, sem).wait()

            def grp(g, c2):
                dst_v = dstb[pl.ds(g * 16, 16)]
                av = alb1[pl.ds(g * 16, 16)]
                ex = jnp.exp(av - gmax)
                dg = plsc.load_gather(den4, [dst_v * 4 + h])
                asm = ex / (dg + 1e-16)
                asmb[pl.ds(g * 16, 16)] = asm
                return c2
            lax.fori_loop(0, _KA // 16, grp, cc)
            pltpu.sync_copy(
                asmb.at[pl.ds(0, _KA)],
                asm_hbm.at[pl.ds(cidx * (HEADS * _KA) + h * _KA, _KA)])

            def scale(e, c2):
                a0 = asmb[pl.ds(e, 16)][0]
                for v in range(4):
                    rows[e, pl.ds(v * 16, 16)] = (rows[e, pl.ds(v * 16, 16)]
                                                  * a0)
                return c2
            lax.fori_loop(0, _KA, scale, cc)
            pltpu.sync_copy(rows, accsh.at[dstb], add=True)
            return cc
        n_chunks = (_NCHUNKS // _NS) + jnp.where(s < _NCHUNKS % _NS, 1, 0)
        lax.fori_loop(0, n_chunks, chunk, 0)

    for p in range(2):
        zero_acc()
        plsc.subcore_barrier()

        @pl.when(c == 0)
        def _(_p=p):
            edge_loop(xqs[2 * _p], 2 * _p)

        @pl.when(c == 1)
        def _(_p=p):
            edge_loop(xqs[2 * _p + 1], 2 * _p + 1)

        plsc.subcore_barrier()

        @pl.when(c == 0)
        def _(_p=p):
            pltpu.sync_copy(accsh.at[pl.ds(s * rows_per, rows_per)],
                            accs[2 * _p].at[pl.ds(s * rows_per, rows_per)])

        @pl.when(c == 1)
        def _(_p=p):
            pltpu.sync_copy(accsh.at[pl.ds(s * rows_per, rows_per)],
                            accs[2 * _p + 1].at[pl.ds(s * rows_per,
                                                      rows_per)])

        plsc.subcore_barrier()


def _pass_c(alpha_f, src, dst, wmax, den_p, xq0, xq1, xq2, xq3):
    mesh = plsc.VectorSubcoreMesh(core_axis_name="c", subcore_axis_name="s",
                                  num_cores=_NC, num_subcores=_NS)
    f = pl.kernel(
        _pc_body,
        out_type=[
            jax.ShapeDtypeStruct((E * HEADS,), jnp.float32),
            jax.ShapeDtypeStruct((N, CH), jnp.float32),
            jax.ShapeDtypeStruct((N, CH), jnp.float32),
            jax.ShapeDtypeStruct((N, CH), jnp.float32),
            jax.ShapeDtypeStruct((N, CH), jnp.float32),
        ],
        mesh=mesh,
        scratch_types=[
            pltpu.VMEM((_NW * HEADS * 16,), jnp.float32),
            pltpu.VMEM((_DENP,), jnp.float32),
            pltpu.VMEM((_DSL,), jnp.float32),
            pltpu.VMEM((_KA,), jnp.float32),
            pltpu.VMEM((_KA + 16,), jnp.float32),
            pltpu.VMEM((_KA,), jnp.int32),
            pltpu.VMEM((_KA,), jnp.int32),
            pltpu.VMEM((_KA, CH), jnp.float32),
            pltpu.VMEM_SHARED((N, CH), jnp.float32),
            pltpu.SemaphoreType.DMA,
            pltpu.SemaphoreType.DMA,
            pltpu.SemaphoreType.DMA,
        ],
        compiler_params=pltpu.CompilerParams(needs_layout_passes=False,
                                             use_tc_tiling_on_sc=False),
    )
    return f(alpha_f, src, dst, wmax, den_p, xq0, xq1, xq2, xq3)


def kernel(x, edge_index, kpts, pts_3d, W_pe1, b_pe1, W_pe2, b_pe2, W_res,
           b_res, W_l, b_l, W_r, b_r, W_e, att, b_conv, ln_g, ln_b, W_proj,
           b_proj):
    x_l, x_r, identity, norm_uv, xq0, xq1, xq2, xq3 = _front(
        x, kpts, pts_3d, W_pe1, b_pe1, W_pe2, b_pe2, W_res, b_res,
        W_l, b_l, W_r, b_r)

    src = edge_index[0]
    dst = edge_index[1]
    alpha_f, ea_f, wmax = _pass_a(x_l, x_r, norm_uv.reshape(-1), src, dst,
                                  W_e.reshape(-1), att.reshape(-1))
    nch = E // _KA
    edge_attr = ea_f.reshape(nch, 3, _KA).transpose(0, 2, 1).reshape(E, 3)

    den_p = _pass_b(alpha_f, dst, wmax)
    asm_f, a0, a1, a2, a3 = _pass_c(alpha_f, src, dst, wmax, den_p,
                                    xq0, xq1, xq2, xq3)
    alpha_sm = asm_f.reshape(nch, HEADS, _KA).transpose(0, 2, 1).reshape(
        E, HEADS)
    acc = jnp.concatenate([a0, a1, a2, a3], axis=1)

    out = _tail(acc, identity, b_conv, ln_g, ln_b, W_proj, b_proj)
    return (out, alpha_sm, edge_attr)


# speedup vs baseline: 13.3124x; 1.0930x over previous
import functools

import jax
import jax.numpy as jnp
from jax import lax
from jax.experimental import pallas as pl
from jax.experimental.pallas import tpu as pltpu
from jax.experimental.pallas import tpu_sc as plsc

N = 10000
E = 160000
D_IN = 256
D_CAT = 320
HID = 256
HEADS = 4
CH = 64

_BN = 1000


def _sigmoid(v):
    return 1.0 / (1.0 + jnp.exp(-v))


def _front_body(x_ref, kpts_ref, pts_ref, wpe1_ref, bpe1_ref, wpe2_ref,
                bpe2_ref, wres_ref, bres_ref, wl_ref, bl_ref, wr_ref, br_ref,
                xl_ref, xr_ref, ident_ref, nuv_ref, xq0_ref, xq1_ref, xq2_ref, xq3_ref):
    kpts = kpts_ref[...]
    nu = kpts[:, 0:1] * (1.0 / 1216.0)
    nv = kpts[:, 1:2] * (1.0 / 352.0)
    depth = pts_ref[:, 2:3]
    w1 = wpe1_ref[...]
    h = (nu * w1[0:1, :] + nv * w1[1:2, :] + depth * w1[2:3, :] + bpe1_ref[...])
    h = h * _sigmoid(h)
    pos = jnp.dot(h, wpe2_ref[...], preferred_element_type=jnp.float32) + bpe2_ref[...]
    x = x_ref[...]

    def lin(w_ref, b_ref):
        w = w_ref[...]
        return (jnp.dot(x, w[:D_IN, :], preferred_element_type=jnp.float32)
                + jnp.dot(pos, w[D_IN:, :], preferred_element_type=jnp.float32)
                + b_ref[...])

    xl = lin(wl_ref, bl_ref)
    xl_ref[...] = xl
    xq0_ref[...] = xl[:, 0:CH]
    xq1_ref[...] = xl[:, CH:2 * CH]
    xq2_ref[...] = xl[:, 2 * CH:3 * CH]
    xq3_ref[...] = xl[:, 3 * CH:]
    xr_ref[...] = lin(wr_ref, br_ref)
    ident_ref[...] = lin(wres_ref, bres_ref)
    nuv_ref[...] = jnp.concatenate([nu, nv], axis=1)


def _front(x, kpts, pts_3d, W_pe1, b_pe1, W_pe2, b_pe2, W_res, b_res,
           W_l, b_l, W_r, b_r):
    grid = (N // _BN,)
    bspec = lambda shp: pl.BlockSpec(shp, lambda i: (i, 0))
    fspec = lambda shp: pl.BlockSpec(shp, lambda i: (0, 0))
    row = lambda d: pl.BlockSpec((d,), lambda i: (0,))
    return pl.pallas_call(
        _front_body,
        grid=grid,
        in_specs=[
            bspec((_BN, D_IN)), bspec((_BN, 2)), bspec((_BN, 3)),
            fspec((3, 32)), row(32), fspec((32, 64)), row(64),
            fspec((D_CAT, HID)), row(HID),
            fspec((D_CAT, HID)), row(HID),
            fspec((D_CAT, HID)), row(HID),
        ],
        out_specs=[
            bspec((_BN, HID)), bspec((_BN, HID)), bspec((_BN, HID)),
            bspec((_BN, 2)), bspec((_BN, CH)), bspec((_BN, CH)),
            bspec((_BN, CH)), bspec((_BN, CH)),
        ],
        out_shape=[
            jax.ShapeDtypeStruct((N, HID), jnp.float32),
            jax.ShapeDtypeStruct((N, HID), jnp.float32),
            jax.ShapeDtypeStruct((N, HID), jnp.float32),
            jax.ShapeDtypeStruct((N, 2), jnp.float32),
            jax.ShapeDtypeStruct((N, CH), jnp.float32),
            jax.ShapeDtypeStruct((N, CH), jnp.float32),
            jax.ShapeDtypeStruct((N, CH), jnp.float32),
            jax.ShapeDtypeStruct((N, CH), jnp.float32),
        ],
    )(x, kpts, pts_3d, W_pe1, b_pe1, W_pe2, b_pe2, W_res, b_res,
      W_l, b_l, W_r, b_r)


def _tail_body(acc_ref, ident_ref, bconv_ref, lng_ref, lnb_ref, wproj_ref,
               bproj_ref, out_ref):
    o = acc_ref[...] + bconv_ref[...]
    mu = jnp.mean(o, axis=-1, keepdims=True)
    var = jnp.mean((o - mu) ** 2, axis=-1, keepdims=True)
    o = (o - mu) / jnp.sqrt(var + 1e-5) * lng_ref[...] + lnb_ref[...]
    o = o * _sigmoid(o)
    o = o + ident_ref[...]
    out_ref[...] = (jnp.dot(o, wproj_ref[...], preferred_element_type=jnp.float32)
                    + bproj_ref[...])


def _tail(acc, ident, b_conv, ln_g, ln_b, W_proj, b_proj):
    grid = (N // _BN,)
    bspec = lambda shp: pl.BlockSpec(shp, lambda i: (i, 0))
    fspec = lambda shp: pl.BlockSpec(shp, lambda i: (0, 0))
    row = lambda d: pl.BlockSpec((d,), lambda i: (0,))
    return pl.pallas_call(
        _tail_body,
        grid=grid,
        in_specs=[
            bspec((_BN, HID)), bspec((_BN, HID)), row(HID),
            row(HID), row(HID), fspec((HID, 256)), row(256),
        ],
        out_specs=pl.BlockSpec((_BN, 256), lambda i: (i, 0)),
        out_shape=jax.ShapeDtypeStruct((N, 256), jnp.float32),
    )(acc, ident, b_conv, ln_g, ln_b, W_proj, b_proj)


_NC = 2
_NS = 16
_NW = _NC * _NS
_EPW = E // _NW
_KA = 128
_NCHUNKS = E // _KA


def _nsqrt(d2):
    i = plsc.bitcast(d2, jnp.int32)
    i = jnp.int32(0x5F3759DF) - lax.shift_right_logical(i, jnp.int32(1))
    y = plsc.bitcast(i, jnp.float32)
    for _ in range(3):
        y = y * (1.5 - 0.5 * d2 * y * y)
    return jnp.where(d2 > 0.0, d2 * y, 0.0)


def _pa_body(xl_hbm, xr_hbm, nuv_hbm, src_hbm, dst_hbm, we_hbm, att_hbm,
             alpha_hbm, ea_hbm, wmax_hbm,
             nuv_v, we_v, att_v, srcb, dstb, xlb, xrb, xf, alb, eab, maxb,
             sem, sem2):
    w = lax.axis_index("s") * _NC + lax.axis_index("c")
    pltpu.sync_copy(nuv_hbm, nuv_v)
    pltpu.sync_copy(we_hbm, we_v)
    pltpu.sync_copy(att_hbm, att_v)
    iota16 = lax.iota(jnp.int32, 16)

    def process_chunk(cidx, wmaxs):
        base = cidx * _KA
        d1 = pltpu.async_copy(src_hbm.at[pl.ds(base, _KA)], srcb, sem)
        d2 = pltpu.async_copy(dst_hbm.at[pl.ds(base, _KA)], dstb, sem2)
        d1.wait()
        d2.wait()
        d1 = pltpu.async_copy(xl_hbm.at[srcb], xlb, sem)
        d2 = pltpu.async_copy(xr_hbm.at[dstb], xrb, sem2)
        d1.wait()
        d2.wait()

        def repack(e, cc):
            ibase = e * 257 + iota16
            for t in range(HID // 16):
                v = xlb[e, pl.ds(t * 16, 16)] + xrb[e, pl.ds(t * 16, 16)]
                plsc.store_scatter(xf, [ibase + t * 16], v)
            return cc
        lax.fori_loop(0, _KA, repack, 0)

        def group_body(g, wm):
            sl = pl.ds(g * 16, 16)
            evec = (iota16 + g * 16) * 257
            src2 = srcb[sl] * 2
            dst2 = dstb[sl] * 2
            u_s = plsc.load_gather(nuv_v, [src2])
            v_s = plsc.load_gather(nuv_v, [src2 + 1])
            u_d = plsc.load_gather(nuv_v, [dst2])
            v_d = plsc.load_gather(nuv_v, [dst2 + 1])
            ru = u_d - u_s
            rv = v_d - v_s
            dist = _nsqrt(ru * ru + rv * rv)
            eab[pl.ds(g * 16, 16)] = ru
            eab[pl.ds(_KA + g * 16, 16)] = rv
            eab[pl.ds(2 * _KA + g * 16, 16)] = dist
            wm_new = []
            for h in range(HEADS):
                def cbody(cb, acc, _h=h):
                    cbase = _h * CH + cb * 16
                    w0v = we_v[pl.ds(cbase, 16)]
                    w1v = we_v[pl.ds(HID + cbase, 16)]
                    w2v = we_v[pl.ds(2 * HID + cbase, 16)]
                    atv = att_v[pl.ds(_h * CH + cb * 16, 16)]
                    for j in range(16):
                        ms = plsc.load_gather(xf, [evec + (cbase + j)])
                        ef = ru * w0v[j] + rv * w1v[j] + dist * w2v[j]
                        m = ms + ef
                        lk = jnp.maximum(m, 0.2 * m)
                        acc = acc + atv[j] * lk
                    return acc
                acc = lax.fori_loop(0, CH // 16, cbody,
                                    jnp.zeros((16,), jnp.float32))
                alb[pl.ds(h * _KA + g * 16, 16)] = acc
                wm_new.append(jnp.maximum(wm[h], acc))
            return tuple(wm_new)

        wmaxs = lax.fori_loop(0, _KA // 16, group_body, wmaxs)
        pltpu.sync_copy(alb, alpha_hbm.at[pl.ds(cidx * (HEADS * _KA),
                                                HEADS * _KA)])
        pltpu.sync_copy(eab, ea_hbm.at[pl.ds(cidx * (3 * _KA), 3 * _KA)])
        return wmaxs

    neg = jnp.full((16,), -jnp.inf, jnp.float32)
    wmaxs = (neg, neg, neg, neg)

    def chunk_body(i, wm):
        return process_chunk(w + i * _NW, wm)

    n_chunks = (_NCHUNKS // _NW) + jnp.where(w < _NCHUNKS % _NW, 1, 0)
    wmaxs = lax.fori_loop(0, n_chunks, chunk_body, wmaxs)
    for h in range(HEADS):
        maxb[pl.ds(h * 16, 16)] = wmaxs[h]
    pltpu.sync_copy(maxb, wmax_hbm.at[pl.ds(w * (HEADS * 16), HEADS * 16)])


def _pass_a(x_l, x_r, nuv_flat, src, dst, we_flat, att_flat):
    mesh = plsc.VectorSubcoreMesh(core_axis_name="c", subcore_axis_name="s",
                                  num_cores=_NC, num_subcores=_NS)
    f = pl.kernel(
        _pa_body,
        out_type=[
            jax.ShapeDtypeStruct((E * HEADS,), jnp.float32),
            jax.ShapeDtypeStruct((E * 3,), jnp.float32),
            jax.ShapeDtypeStruct((_NW * HEADS * 16,), jnp.float32),
        ],
        mesh=mesh,
        scratch_types=[
            pltpu.VMEM((2 * N,), jnp.float32),
            pltpu.VMEM((3 * HID,), jnp.float32),
            pltpu.VMEM((HEADS * CH,), jnp.float32),
            pltpu.VMEM((_KA,), jnp.int32),
            pltpu.VMEM((_KA,), jnp.int32),
            pltpu.VMEM((_KA, HID), jnp.float32),
            pltpu.VMEM((_KA, HID), jnp.float32),
            pltpu.VMEM((_KA * 257,), jnp.float32),
            pltpu.VMEM((HEADS * _KA,), jnp.float32),
            pltpu.VMEM((3 * _KA,), jnp.float32),
            pltpu.VMEM((HEADS * 16,), jnp.float32),
            pltpu.SemaphoreType.DMA,
            pltpu.SemaphoreType.DMA,
        ],
        compiler_params=pltpu.CompilerParams(needs_layout_passes=False,
                                             use_tc_tiling_on_sc=False),
    )
    return f(x_l, x_r, nuv_flat, src, dst, we_flat, att_flat)


_DENP = 40960
_DSL = _DENP // _NS


def _pb_body(alpha_hbm, dst_hbm, wmax_hbm, den_hbm,
             wmv, alb, dstb, den4, tmpv, shden, sem):
    c = lax.axis_index("c")
    s = lax.axis_index("s")
    w = s * _NC + c
    pltpu.sync_copy(wmax_hbm, wmv)

    def mxb(i, m):
        return jnp.maximum(m, wmv[pl.ds(i * 16, 16)])
    gv = lax.fori_loop(0, (_NW * HEADS * 16) // 16, mxb,
                       jnp.full((16,), -jnp.inf, jnp.float32))
    gmax = jnp.max(gv)

    def zb(i, cc):
        den4[pl.ds(i * 16, 16)] = jnp.zeros((16,), jnp.float32)
        return cc
    lax.fori_loop(0, _DENP // 16, zb, 0)

    def chunk(i, cc):
        cidx = w + i * _NW
        pltpu.sync_copy(dst_hbm.at[pl.ds(cidx * _KA, _KA)], dstb)
        pltpu.sync_copy(alpha_hbm.at[pl.ds(cidx * (HEADS * _KA),
                                           HEADS * _KA)], alb)

        def grp(g, c2):
            d4 = dstb[pl.ds(g * 16, 16)] * 4
            for h in range(HEADS):
                v = jnp.exp(alb[pl.ds(h * _KA + g * 16, 16)] - gmax)
                plsc.addupdate_scatter(den4, [d4 + h], v)
            return c2
        return lax.fori_loop(0, _KA // 16, grp, cc)

    n_chunks = (_NCHUNKS // _NW) + jnp.where(w < _NCHUNKS % _NW, 1, 0)
    lax.fori_loop(0, n_chunks, chunk, 0)

    pltpu.sync_copy(den4, shden.at[s])
    plsc.subcore_barrier()
    off = s * _DSL
    pltpu.sync_copy(shden.at[0, pl.ds(off, _DSL)], den4.at[pl.ds(0, _DSL)])

    def redt(t, cc):
        pltpu.sync_copy(shden.at[t, pl.ds(off, _DSL)], tmpv)

        def addv(i, c3):
            den4[pl.ds(i * 16, 16)] = (den4[pl.ds(i * 16, 16)]
                                       + tmpv[pl.ds(i * 16, 16)])
            return c3
        return lax.fori_loop(0, _DSL // 16, addv, cc)
    lax.fori_loop(1, _NS, redt, 0)
    pltpu.sync_copy(den4.at[pl.ds(0, _DSL)],
                    den_hbm.at[c, pl.ds(off, _DSL)])


def _pass_b(alpha_f, dst, wmax):
    mesh = plsc.VectorSubcoreMesh(core_axis_name="c", subcore_axis_name="s",
                                  num_cores=_NC, num_subcores=_NS)
    f = pl.kernel(
        _pb_body,
        out_type=[jax.ShapeDtypeStruct((_NC, _DENP), jnp.float32)],
        mesh=mesh,
        scratch_types=[
            pltpu.VMEM((_NW * HEADS * 16,), jnp.float32),
            pltpu.VMEM((HEADS * _KA,), jnp.float32),
            pltpu.VMEM((_KA,), jnp.int32),
            pltpu.VMEM((_DENP,), jnp.float32),
            pltpu.VMEM((_DSL,), jnp.float32),
            pltpu.VMEM_SHARED((_NS, _DENP), jnp.float32),
            pltpu.SemaphoreType.DMA,
        ],
        compiler_params=pltpu.CompilerParams(needs_layout_passes=False,
                                             use_tc_tiling_on_sc=False),
    )
    return f(alpha_f, dst, wmax)[0]


def _pc_body(alpha_hbm, src_hbm, dst_hbm, wmax_hbm, den_hbm,
             xq0_hbm, xq1_hbm, xq2_hbm, xq3_hbm,
             asm_hbm, a0_hbm, a1_hbm, a2_hbm, a3_hbm,
             wmv, den4, tmpv, alb1, alb1b, asmb, srcb, srcb2, dstb, dstb2,
             rows, rowsb, accsh, sem, sem2, sem3, semb, sem2b, sem3b, semg,
             semgb):
    c = lax.axis_index("c")
    s = lax.axis_index("s")
    pltpu.sync_copy(wmax_hbm, wmv)

    def mxb(i, m):
        return jnp.maximum(m, wmv[pl.ds(i * 16, 16)])
    gv = lax.fori_loop(0, (_NW * HEADS * 16) // 16, mxb,
                       jnp.full((16,), -jnp.inf, jnp.float32))
    gmax = jnp.max(gv)

    pltpu.sync_copy(den_hbm.at[0], den4)

    def dsum(i, cc):
        pltpu.sync_copy(den_hbm.at[1, pl.ds(i * _DSL, _DSL)], tmpv)

        def addv(j, c3):
            den4[pl.ds(i * _DSL + j * 16, 16)] = (
                den4[pl.ds(i * _DSL + j * 16, 16)] + tmpv[pl.ds(j * 16, 16)])
            return c3
        return lax.fori_loop(0, _DSL // 16, addv, cc)
    lax.fori_loop(0, _NS, dsum, 0)

    rows_per = N // _NS
    xqs = (xq0_hbm, xq1_hbm, xq2_hbm, xq3_hbm)
    accs = (a0_hbm, a1_hbm, a2_hbm, a3_hbm)

    def zero_acc():
        def zrow(i, cc):
            for v in range(4):
                rows[i, pl.ds(v * 16, 16)] = jnp.zeros((16,), jnp.float32)
            return cc
        lax.fori_loop(0, _KA, zrow, 0)

        def zacc(i, cc):
            pltpu.sync_copy(rows,
                            accsh.at[pl.ds(s * rows_per + i * _KA, _KA)])
            return cc
        lax.fori_loop(0, rows_per // _KA, zacc, 0)
        pltpu.sync_copy(
            rows.at[pl.ds(0, rows_per % _KA)],
            accsh.at[pl.ds(s * rows_per + (rows_per // _KA) * _KA,
                           rows_per % _KA)])

    def edge_loop(xq_hbm, h):
        n_chunks = (_NCHUNKS // _NS) + jnp.where(s < _NCHUNKS % _NS, 1, 0)
        srcs = (srcb, srcb2)
        dsts = (dstb, dstb2)
        albs = (alb1, alb1b)
        rowss = (rows, rowsb)
        sems_i = ((sem, sem2, sem3), (semb, sem2b, sem3b))
        sems_g = (semg, semgb)

        def issue_idx(i, sl):
            cidx = s + i * _NS
            base = cidx * _KA
            pltpu.async_copy(src_hbm.at[pl.ds(base, _KA)], srcs[sl],
                             sems_i[sl][0])
            pltpu.async_copy(dst_hbm.at[pl.ds(base, _KA)], dsts[sl],
                             sems_i[sl][1])
            pltpu.async_copy(
                alpha_hbm.at[pl.ds(cidx * (HEADS * _KA) + h * _KA, _KA)],
                albs[sl], sems_i[sl][2])

        def wait_idx(sl):
            pltpu.make_async_copy(src_hbm.at[pl.ds(0, _KA)], srcs[sl],
                                  sems_i[sl][0]).wait()
            pltpu.make_async_copy(dst_hbm.at[pl.ds(0, _KA)], dsts[sl],
                                  sems_i[sl][1]).wait()
            pltpu.make_async_copy(
                alpha_hbm.at[pl.ds(0, _KA)], albs[sl], sems_i[sl][2]).wait()

        def issue_gather(sl):
            pltpu.async_copy(xq_hbm.at[srcs[sl]], rowss[sl], sems_g[sl])

        def wait_gather(sl):
            pltpu.make_async_copy(xq_hbm.at[srcs[sl]], rowss[sl],
                                  sems_g[sl]).wait()

        issue_idx(0, 0)
        wait_idx(0)
        issue_gather(0)

        @pl.when(n_chunks > 1)
        def _():
            issue_idx(1, 1)

        def chunk(i, cc):
            cidx = s + i * _NS
            sl = lax.rem(i, 2)

            def do_slot(sl_s):
                srcb_s = srcs[sl_s]
                dstb_s = dsts[sl_s]
                alb_s = albs[sl_s]
                rows_s = rowss[sl_s]
                wait_gather(sl_s)

                @pl.when(i + 1 < n_chunks)
                def _():
                    wait_idx(1 - sl_s)
                    issue_gather(1 - sl_s)

                def grp(g, c2):
                    dst_v = dstb_s[pl.ds(g * 16, 16)]
                    av = alb_s[pl.ds(g * 16, 16)]
                    ex = jnp.exp(av - gmax)
                    dg = plsc.load_gather(den4, [dst_v * 4 + h])
                    asm = ex / (dg + 1e-16)
                    asmb[pl.ds(g * 16, 16)] = asm
                    return c2
                lax.fori_loop(0, _KA // 16, grp, 0)
                pltpu.sync_copy(
                    asmb.at[pl.ds(0, _KA)],
                    asm_hbm.at[pl.ds(cidx * (HEADS * _KA) + h * _KA, _KA)])

                def scale(e, c2):
                    a0 = asmb[pl.ds(e, 16)][0]
                    for v in range(4):
                        rows_s[e, pl.ds(v * 16, 16)] = (
                            rows_s[e, pl.ds(v * 16, 16)] * a0)
                    return c2
                lax.fori_loop(0, _KA, scale, 0)
                pltpu.sync_copy(rows_s, accsh.at[dstb_s], add=True)

                @pl.when(i + 2 < n_chunks)
                def _():
                    issue_idx(i + 2, sl_s)

            @pl.when(sl == 0)
            def _():
                do_slot(0)

            @pl.when(sl == 1)
            def _():
                do_slot(1)
            return cc
        lax.fori_loop(0, n_chunks, chunk, 0)

    for p in range(2):
        zero_acc()
        plsc.subcore_barrier()

        @pl.when(c == 0)
        def _(_p=p):
            edge_loop(xqs[2 * _p], 2 * _p)

        @pl.when(c == 1)
        def _(_p=p):
            edge_loop(xqs[2 * _p + 1], 2 * _p + 1)

        plsc.subcore_barrier()

        @pl.when(c == 0)
        def _(_p=p):
            pltpu.sync_copy(accsh.at[pl.ds(s * rows_per, rows_per)],
                            accs[2 * _p].at[pl.ds(s * rows_per, rows_per)])

        @pl.when(c == 1)
        def _(_p=p):
            pltpu.sync_copy(accsh.at[pl.ds(s * rows_per, rows_per)],
                            accs[2 * _p + 1].at[pl.ds(s * rows_per,
                                                      rows_per)])

        plsc.subcore_barrier()


def _pass_c(alpha_f, src, dst, wmax, den_p, xq0, xq1, xq2, xq3):
    mesh = plsc.VectorSubcoreMesh(core_axis_name="c", subcore_axis_name="s",
                                  num_cores=_NC, num_subcores=_NS)
    f = pl.kernel(
        _pc_body,
        out_type=[
            jax.ShapeDtypeStruct((E * HEADS,), jnp.float32),
            jax.ShapeDtypeStruct((N, CH), jnp.float32),
            jax.ShapeDtypeStruct((N, CH), jnp.float32),
            jax.ShapeDtypeStruct((N, CH), jnp.float32),
            jax.ShapeDtypeStruct((N, CH), jnp.float32),
        ],
        mesh=mesh,
        scratch_types=[
            pltpu.VMEM((_NW * HEADS * 16,), jnp.float32),
            pltpu.VMEM((_DENP,), jnp.float32),
            pltpu.VMEM((_DSL,), jnp.float32),
            pltpu.VMEM((_KA,), jnp.float32),
            pltpu.VMEM((_KA,), jnp.float32),
            pltpu.VMEM((_KA + 16,), jnp.float32),
            pltpu.VMEM((_KA,), jnp.int32),
            pltpu.VMEM((_KA,), jnp.int32),
            pltpu.VMEM((_KA,), jnp.int32),
            pltpu.VMEM((_KA,), jnp.int32),
            pltpu.VMEM((_KA, CH), jnp.float32),
            pltpu.VMEM((_KA, CH), jnp.float32),
            pltpu.VMEM_SHARED((N, CH), jnp.float32),
            pltpu.SemaphoreType.DMA,
            pltpu.SemaphoreType.DMA,
            pltpu.SemaphoreType.DMA,
            pltpu.SemaphoreType.DMA,
            pltpu.SemaphoreType.DMA,
            pltpu.SemaphoreType.DMA,
            pltpu.SemaphoreType.DMA,
            pltpu.SemaphoreType.DMA,
        ],
        compiler_params=pltpu.CompilerParams(needs_layout_passes=False,
                                             use_tc_tiling_on_sc=False),
    )
    return f(alpha_f, src, dst, wmax, den_p, xq0, xq1, xq2, xq3)


def kernel(x, edge_index, kpts, pts_3d, W_pe1, b_pe1, W_pe2, b_pe2, W_res,
           b_res, W_l, b_l, W_r, b_r, W_e, att, b_conv, ln_g, ln_b, W_proj,
           b_proj):
    x_l, x_r, identity, norm_uv, xq0, xq1, xq2, xq3 = _front(
        x, kpts, pts_3d, W_pe1, b_pe1, W_pe2, b_pe2, W_res, b_res,
        W_l, b_l, W_r, b_r)

    src = edge_index[0]
    dst = edge_index[1]
    alpha_f, ea_f, wmax = _pass_a(x_l, x_r, norm_uv.reshape(-1), src, dst,
                                  W_e.reshape(-1), att.reshape(-1))
    nch = E // _KA
    edge_attr = ea_f.reshape(nch, 3, _KA).transpose(0, 2, 1).reshape(E, 3)

    den_p = _pass_b(alpha_f, dst, wmax)
    asm_f, a0, a1, a2, a3 = _pass_c(alpha_f, src, dst, wmax, den_p,
                                    xq0, xq1, xq2, xq3)
    alpha_sm = asm_f.reshape(nch, HEADS, _KA).transpose(0, 2, 1).reshape(
        E, HEADS)
    acc = jnp.concatenate([a0, a1, a2, a3], axis=1)

    out = _tail(acc, identity, b_conv, ln_g, ln_b, W_proj, b_proj)
    return (out, alpha_sm, edge_attr)


# speedup vs baseline: 14.7342x; 1.1068x over previous
import functools

import jax
import jax.numpy as jnp
from jax import lax
from jax.experimental import pallas as pl
from jax.experimental.pallas import tpu as pltpu
from jax.experimental.pallas import tpu_sc as plsc

N = 10000
E = 160000
D_IN = 256
D_CAT = 320
HID = 256
HEADS = 4
CH = 64

_BN = 1000


def _sigmoid(v):
    return 1.0 / (1.0 + jnp.exp(-v))


def _front_body(x_ref, kpts_ref, pts_ref, wpe1_ref, bpe1_ref, wpe2_ref,
                bpe2_ref, wres_ref, bres_ref, wl_ref, bl_ref, wr_ref, br_ref,
                xl_ref, xr_ref, ident_ref, nuv_ref, xq0_ref, xq1_ref, xq2_ref, xq3_ref):
    kpts = kpts_ref[...]
    nu = kpts[:, 0:1] * (1.0 / 1216.0)
    nv = kpts[:, 1:2] * (1.0 / 352.0)
    depth = pts_ref[:, 2:3]
    w1 = wpe1_ref[...]
    h = (nu * w1[0:1, :] + nv * w1[1:2, :] + depth * w1[2:3, :] + bpe1_ref[...])
    h = h * _sigmoid(h)
    pos = jnp.dot(h, wpe2_ref[...], preferred_element_type=jnp.float32) + bpe2_ref[...]
    x = x_ref[...]

    def lin(w_ref, b_ref):
        w = w_ref[...]
        return (jnp.dot(x, w[:D_IN, :], preferred_element_type=jnp.float32)
                + jnp.dot(pos, w[D_IN:, :], preferred_element_type=jnp.float32)
                + b_ref[...])

    xl = lin(wl_ref, bl_ref)
    xl_ref[...] = xl
    xq0_ref[...] = xl[:, 0:CH]
    xq1_ref[...] = xl[:, CH:2 * CH]
    xq2_ref[...] = xl[:, 2 * CH:3 * CH]
    xq3_ref[...] = xl[:, 3 * CH:]
    xr_ref[...] = lin(wr_ref, br_ref)
    ident_ref[...] = lin(wres_ref, bres_ref)
    nuv_ref[...] = jnp.concatenate([nu, nv], axis=1)


def _front(x, kpts, pts_3d, W_pe1, b_pe1, W_pe2, b_pe2, W_res, b_res,
           W_l, b_l, W_r, b_r):
    grid = (N // _BN,)
    bspec = lambda shp: pl.BlockSpec(shp, lambda i: (i, 0))
    fspec = lambda shp: pl.BlockSpec(shp, lambda i: (0, 0))
    row = lambda d: pl.BlockSpec((d,), lambda i: (0,))
    return pl.pallas_call(
        _front_body,
        grid=grid,
        in_specs=[
            bspec((_BN, D_IN)), bspec((_BN, 2)), bspec((_BN, 3)),
            fspec((3, 32)), row(32), fspec((32, 64)), row(64),
            fspec((D_CAT, HID)), row(HID),
            fspec((D_CAT, HID)), row(HID),
            fspec((D_CAT, HID)), row(HID),
        ],
        out_specs=[
            bspec((_BN, HID)), bspec((_BN, HID)), bspec((_BN, HID)),
            bspec((_BN, 2)), bspec((_BN, CH)), bspec((_BN, CH)),
            bspec((_BN, CH)), bspec((_BN, CH)),
        ],
        out_shape=[
            jax.ShapeDtypeStruct((N, HID), jnp.float32),
            jax.ShapeDtypeStruct((N, HID), jnp.float32),
            jax.ShapeDtypeStruct((N, HID), jnp.float32),
            jax.ShapeDtypeStruct((N, 2), jnp.float32),
            jax.ShapeDtypeStruct((N, CH), jnp.float32),
            jax.ShapeDtypeStruct((N, CH), jnp.float32),
            jax.ShapeDtypeStruct((N, CH), jnp.float32),
            jax.ShapeDtypeStruct((N, CH), jnp.float32),
        ],
    )(x, kpts, pts_3d, W_pe1, b_pe1, W_pe2, b_pe2, W_res, b_res,
      W_l, b_l, W_r, b_r)


def _tail_body(acc_ref, ident_ref, bconv_ref, lng_ref, lnb_ref, wproj_ref,
               bproj_ref, out_ref):
    o = acc_ref[...] + bconv_ref[...]
    mu = jnp.mean(o, axis=-1, keepdims=True)
    var = jnp.mean((o - mu) ** 2, axis=-1, keepdims=True)
    o = (o - mu) / jnp.sqrt(var + 1e-5) * lng_ref[...] + lnb_ref[...]
    o = o * _sigmoid(o)
    o = o + ident_ref[...]
    out_ref[...] = (jnp.dot(o, wproj_ref[...], preferred_element_type=jnp.float32)
                    + bproj_ref[...])


def _tail(acc, ident, b_conv, ln_g, ln_b, W_proj, b_proj):
    grid = (N // _BN,)
    bspec = lambda shp: pl.BlockSpec(shp, lambda i: (i, 0))
    fspec = lambda shp: pl.BlockSpec(shp, lambda i: (0, 0))
    row = lambda d: pl.BlockSpec((d,), lambda i: (0,))
    return pl.pallas_call(
        _tail_body,
        grid=grid,
        in_specs=[
            bspec((_BN, HID)), bspec((_BN, HID)), row(HID),
            row(HID), row(HID), fspec((HID, 256)), row(256),
        ],
        out_specs=pl.BlockSpec((_BN, 256), lambda i: (i, 0)),
        out_shape=jax.ShapeDtypeStruct((N, 256), jnp.float32),
    )(acc, ident, b_conv, ln_g, ln_b, W_proj, b_proj)


_NC = 2
_NS = 16
_NW = _NC * _NS
_EPW = E // _NW
_KA = 128
_NCHUNKS = E // _KA


def _nsqrt(d2):
    i = plsc.bitcast(d2, jnp.int32)
    i = jnp.int32(0x5F3759DF) - lax.shift_right_logical(i, jnp.int32(1))
    y = plsc.bitcast(i, jnp.float32)
    for _ in range(3):
        y = y * (1.5 - 0.5 * d2 * y * y)
    return jnp.where(d2 > 0.0, d2 * y, 0.0)


def _pa_body(xl_hbm, xr_hbm, nuv_hbm, src_hbm, dst_hbm, we_hbm, att_hbm,
             alpha_hbm, ea_hbm, wmax_hbm,
             nuv_v, we_v, att_v, srcb, dstb, srcb2, dstb2, xlb, xrb, xf, alb,
             eab, maxb, sem, sem2, semi, semi2):
    w = lax.axis_index("s") * _NC + lax.axis_index("c")
    pltpu.sync_copy(nuv_hbm, nuv_v)
    pltpu.sync_copy(we_hbm, we_v)
    pltpu.sync_copy(att_hbm, att_v)
    iota16 = lax.iota(jnp.int32, 16)

    srcs = (srcb, srcb2)
    dsts = (dstb, dstb2)

    def issue_idx(cidx, sl):
        base = cidx * _KA
        pltpu.async_copy(src_hbm.at[pl.ds(base, _KA)], srcs[sl], semi)
        pltpu.async_copy(dst_hbm.at[pl.ds(base, _KA)], dsts[sl], semi2)

    def wait_idx(sl):
        pltpu.make_async_copy(src_hbm.at[pl.ds(0, _KA)], srcs[sl],
                              semi).wait()
        pltpu.make_async_copy(dst_hbm.at[pl.ds(0, _KA)], dsts[sl],
                              semi2).wait()

    def issue_rows(sl):
        pltpu.async_copy(xl_hbm.at[srcs[sl]], xlb, sem)
        pltpu.async_copy(xr_hbm.at[dsts[sl]], xrb, sem2)

    def wait_rows(sl):
        pltpu.make_async_copy(xl_hbm.at[srcs[sl]], xlb, sem).wait()
        pltpu.make_async_copy(xr_hbm.at[dsts[sl]], xrb, sem2).wait()

    def process_chunk(cidx, next_cidx, sl, has_next, wmaxs):
        wait_rows(sl)

        def repack(e, cc):
            ibase = e * 257 + iota16
            for t in range(HID // 16):
                v = xlb[e, pl.ds(t * 16, 16)] + xrb[e, pl.ds(t * 16, 16)]
                plsc.store_scatter(xf, [ibase + t * 16], v)
            return cc
        lax.fori_loop(0, _KA, repack, 0)

        @pl.when(has_next)
        def _():
            wait_idx(1 - sl)
            issue_rows(1 - sl)

        def group_body(g, wm):
            gsl = pl.ds(g * 16, 16)
            evec = (iota16 + g * 16) * 257
            src2 = srcs[sl][gsl] * 2
            dst2 = dsts[sl][gsl] * 2
            u_s = plsc.load_gather(nuv_v, [src2])
            v_s = plsc.load_gather(nuv_v, [src2 + 1])
            u_d = plsc.load_gather(nuv_v, [dst2])
            v_d = plsc.load_gather(nuv_v, [dst2 + 1])
            ru = u_d - u_s
            rv = v_d - v_s
            dist = _nsqrt(ru * ru + rv * rv)
            eab[pl.ds(g * 16, 16)] = ru
            eab[pl.ds(_KA + g * 16, 16)] = rv
            eab[pl.ds(2 * _KA + g * 16, 16)] = dist
            for h in range(HEADS):
                def cbody(cb, acc, _h=h):
                    cbase = _h * CH + cb * 16
                    w0v = we_v[pl.ds(cbase, 16)]
                    w1v = we_v[pl.ds(HID + cbase, 16)]
                    w2v = we_v[pl.ds(2 * HID + cbase, 16)]
                    atv = att_v[pl.ds(_h * CH + cb * 16, 16)]
                    for j in range(16):
                        ms = plsc.load_gather(xf, [evec + (cbase + j)])
                        ef = ru * w0v[j] + rv * w1v[j] + dist * w2v[j]
                        m = ms + ef
                        lk = jnp.maximum(m, 0.2 * m)
                        acc = acc + atv[j] * lk
                    return acc
                acc = lax.fori_loop(0, CH // 16, cbody,
                                    jnp.zeros((16,), jnp.float32))
                alb[pl.ds(h * _KA + g * 16, 16)] = acc
            return wm

        lax.fori_loop(0, _KA // 16, group_body, 0)

        @pl.when(jnp.logical_and(has_next, next_cidx >= 0))
        def _():
            issue_idx(next_cidx, sl)

        pltpu.sync_copy(alb, alpha_hbm.at[pl.ds(cidx * (HEADS * _KA),
                                                HEADS * _KA)])
        pltpu.sync_copy(eab, ea_hbm.at[pl.ds(cidx * (3 * _KA), 3 * _KA)])

    neg = jnp.full((16,), -jnp.inf, jnp.float32)
    wmaxs = (neg, neg, neg, neg)
    n_chunks = (_NCHUNKS // _NW) + jnp.where(w < _NCHUNKS % _NW, 1, 0)

    issue_idx(w, 0)
    wait_idx(0)
    issue_rows(0)

    @pl.when(n_chunks > 1)
    def _():
        issue_idx(w + _NW, 1)

    def chunk_body(i, wm):
        cidx = w + i * _NW
        next2 = jnp.where(i + 2 < n_chunks, w + (i + 2) * _NW,
                          jnp.int32(-1))
        has_next = i + 1 < n_chunks
        slp = lax.rem(i, 2)

        @pl.when(slp == 0)
        def _():
            process_chunk(cidx, next2, 0, has_next, wm)

        @pl.when(slp == 1)
        def _():
            process_chunk(cidx, next2, 1, has_next, wm)

        wm_new = []
        for h in range(HEADS):
            def mgb(g, m, _h=h):
                return jnp.maximum(m, alb[pl.ds(_h * _KA + g * 16, 16)])
            wm_new.append(lax.fori_loop(0, _KA // 16, mgb, wm[h]))
        return tuple(wm_new)
    wmaxs = lax.fori_loop(0, n_chunks, chunk_body, wmaxs)
    for h in range(HEADS):
        maxb[pl.ds(h * 16, 16)] = wmaxs[h]
    pltpu.sync_copy(maxb, wmax_hbm.at[pl.ds(w * (HEADS * 16), HEADS * 16)])


def _pass_a(x_l, x_r, nuv_flat, src, dst, we_flat, att_flat):
    mesh = plsc.VectorSubcoreMesh(core_axis_name="c", subcore_axis_name="s",
                                  num_cores=_NC, num_subcores=_NS)
    f = pl.kernel(
        _pa_body,
        out_type=[
            jax.ShapeDtypeStruct((E * HEADS,), jnp.float32),
            jax.ShapeDtypeStruct((E * 3,), jnp.float32),
            jax.ShapeDtypeStruct((_NW * HEADS * 16,), jnp.float32),
        ],
        mesh=mesh,
        scratch_types=[
            pltpu.VMEM((2 * N,), jnp.float32),
            pltpu.VMEM((3 * HID,), jnp.float32),
            pltpu.VMEM((HEADS * CH,), jnp.float32),
            pltpu.VMEM((_KA,), jnp.int32),
            pltpu.VMEM((_KA,), jnp.int32),
            pltpu.VMEM((_KA,), jnp.int32),
            pltpu.VMEM((_KA,), jnp.int32),
            pltpu.VMEM((_KA, HID), jnp.float32),
            pltpu.VMEM((_KA, HID), jnp.float32),
            pltpu.VMEM((_KA * 257,), jnp.float32),
            pltpu.VMEM((HEADS * _KA,), jnp.float32),
            pltpu.VMEM((3 * _KA,), jnp.float32),
            pltpu.VMEM((HEADS * 16,), jnp.float32),
            pltpu.SemaphoreType.DMA,
            pltpu.SemaphoreType.DMA,
            pltpu.SemaphoreType.DMA,
            pltpu.SemaphoreType.DMA,
        ],
        compiler_params=pltpu.CompilerParams(needs_layout_passes=False,
                                             use_tc_tiling_on_sc=False),
    )
    return f(x_l, x_r, nuv_flat, src, dst, we_flat, att_flat)


_DENP = 40960
_DSL = _DENP // _NS


def _pb_body(alpha_hbm, dst_hbm, wmax_hbm, den_hbm,
             wmv, alb, dstb, den4, tmpv, shden, sem):
    c = lax.axis_index("c")
    s = lax.axis_index("s")
    w = s * _NC + c
    pltpu.sync_copy(wmax_hbm, wmv)

    def mxb(i, m):
        return jnp.maximum(m, wmv[pl.ds(i * 16, 16)])
    gv = lax.fori_loop(0, (_NW * HEADS * 16) // 16, mxb,
                       jnp.full((16,), -jnp.inf, jnp.float32))
    gmax = jnp.max(gv)

    def zb(i, cc):
        den4[pl.ds(i * 16, 16)] = jnp.zeros((16,), jnp.float32)
        return cc
    lax.fori_loop(0, _DENP // 16, zb, 0)

    def chunk(i, cc):
        cidx = w + i * _NW
        pltpu.sync_copy(dst_hbm.at[pl.ds(cidx * _KA, _KA)], dstb)
        pltpu.sync_copy(alpha_hbm.at[pl.ds(cidx * (HEADS * _KA),
                                           HEADS * _KA)], alb)

        def grp(g, c2):
            d4 = dstb[pl.ds(g * 16, 16)] * 4
            for h in range(HEADS):
                v = jnp.exp(alb[pl.ds(h * _KA + g * 16, 16)] - gmax)
                plsc.addupdate_scatter(den4, [d4 + h], v)
            return c2
        return lax.fori_loop(0, _KA // 16, grp, cc)

    n_chunks = (_NCHUNKS // _NW) + jnp.where(w < _NCHUNKS % _NW, 1, 0)
    lax.fori_loop(0, n_chunks, chunk, 0)

    pltpu.sync_copy(den4, shden.at[s])
    plsc.subcore_barrier()
    off = s * _DSL
    pltpu.sync_copy(shden.at[0, pl.ds(off, _DSL)], den4.at[pl.ds(0, _DSL)])

    def redt(t, cc):
        pltpu.sync_copy(shden.at[t, pl.ds(off, _DSL)], tmpv)

        def addv(i, c3):
            den4[pl.ds(i * 16, 16)] = (den4[pl.ds(i * 16, 16)]
                                       + tmpv[pl.ds(i * 16, 16)])
            return c3
        return lax.fori_loop(0, _DSL // 16, addv, cc)
    lax.fori_loop(1, _NS, redt, 0)
    pltpu.sync_copy(den4.at[pl.ds(0, _DSL)],
                    den_hbm.at[c, pl.ds(off, _DSL)])


def _pass_b(alpha_f, dst, wmax):
    mesh = plsc.VectorSubcoreMesh(core_axis_name="c", subcore_axis_name="s",
                                  num_cores=_NC, num_subcores=_NS)
    f = pl.kernel(
        _pb_body,
        out_type=[jax.ShapeDtypeStruct((_NC, _DENP), jnp.float32)],
        mesh=mesh,
        scratch_types=[
            pltpu.VMEM((_NW * HEADS * 16,), jnp.float32),
            pltpu.VMEM((HEADS * _KA,), jnp.float32),
            pltpu.VMEM((_KA,), jnp.int32),
            pltpu.VMEM((_DENP,), jnp.float32),
            pltpu.VMEM((_DSL,), jnp.float32),
            pltpu.VMEM_SHARED((_NS, _DENP), jnp.float32),
            pltpu.SemaphoreType.DMA,
        ],
        compiler_params=pltpu.CompilerParams(needs_layout_passes=False,
                                             use_tc_tiling_on_sc=False),
    )
    return f(alpha_f, dst, wmax)[0]


def _pc_body(alpha_hbm, src_hbm, dst_hbm, wmax_hbm, den_hbm,
             xq0_hbm, xq1_hbm, xq2_hbm, xq3_hbm,
             asm_hbm, a0_hbm, a1_hbm, a2_hbm, a3_hbm,
             wmv, den4, tmpv, alb1, alb1b, asmb, srcb, srcb2, dstb, dstb2,
             rows, rowsb, accsh, sem, sem2, sem3, semb, sem2b, sem3b, semg,
             semgb):
    c = lax.axis_index("c")
    s = lax.axis_index("s")
    pltpu.sync_copy(wmax_hbm, wmv)

    def mxb(i, m):
        return jnp.maximum(m, wmv[pl.ds(i * 16, 16)])
    gv = lax.fori_loop(0, (_NW * HEADS * 16) // 16, mxb,
                       jnp.full((16,), -jnp.inf, jnp.float32))
    gmax = jnp.max(gv)

    pltpu.sync_copy(den_hbm.at[0], den4)

    def dsum(i, cc):
        pltpu.sync_copy(den_hbm.at[1, pl.ds(i * _DSL, _DSL)], tmpv)

        def addv(j, c3):
            den4[pl.ds(i * _DSL + j * 16, 16)] = (
                den4[pl.ds(i * _DSL + j * 16, 16)] + tmpv[pl.ds(j * 16, 16)])
            return c3
        return lax.fori_loop(0, _DSL // 16, addv, cc)
    lax.fori_loop(0, _NS, dsum, 0)

    rows_per = N // _NS
    xqs = (xq0_hbm, xq1_hbm, xq2_hbm, xq3_hbm)
    accs = (a0_hbm, a1_hbm, a2_hbm, a3_hbm)

    def zero_acc():
        def zrow(i, cc):
            for v in range(4):
                rows[i, pl.ds(v * 16, 16)] = jnp.zeros((16,), jnp.float32)
            return cc
        lax.fori_loop(0, _KA, zrow, 0)

        def zacc(i, cc):
            pltpu.sync_copy(rows,
                            accsh.at[pl.ds(s * rows_per + i * _KA, _KA)])
            return cc
        lax.fori_loop(0, rows_per // _KA, zacc, 0)
        pltpu.sync_copy(
            rows.at[pl.ds(0, rows_per % _KA)],
            accsh.at[pl.ds(s * rows_per + (rows_per // _KA) * _KA,
                           rows_per % _KA)])

    def edge_loop(xq_hbm, h):
        n_chunks = (_NCHUNKS // _NS) + jnp.where(s < _NCHUNKS % _NS, 1, 0)
        srcs = (srcb, srcb2)
        dsts = (dstb, dstb2)
        albs = (alb1, alb1b)
        rowss = (rows, rowsb)
        sems_i = ((sem, sem2, sem3), (semb, sem2b, sem3b))
        sems_g = (semg, semgb)

        def issue_idx(i, sl):
            cidx = s + i * _NS
            base = cidx * _KA
            pltpu.async_copy(src_hbm.at[pl.ds(base, _KA)], srcs[sl],
                             sems_i[sl][0])
            pltpu.async_copy(dst_hbm.at[pl.ds(base, _KA)], dsts[sl],
                             sems_i[sl][1])
            pltpu.async_copy(
                alpha_hbm.at[pl.ds(cidx * (HEADS * _KA) + h * _KA, _KA)],
                albs[sl], sems_i[sl][2])

        def wait_idx(sl):
            pltpu.make_async_copy(src_hbm.at[pl.ds(0, _KA)], srcs[sl],
                                  sems_i[sl][0]).wait()
            pltpu.make_async_copy(dst_hbm.at[pl.ds(0, _KA)], dsts[sl],
                                  sems_i[sl][1]).wait()
            pltpu.make_async_copy(
                alpha_hbm.at[pl.ds(0, _KA)], albs[sl], sems_i[sl][2]).wait()

        def issue_gather(sl):
            pltpu.async_copy(xq_hbm.at[srcs[sl]], rowss[sl], sems_g[sl])

        def wait_gather(sl):
            pltpu.make_async_copy(xq_hbm.at[srcs[sl]], rowss[sl],
                                  sems_g[sl]).wait()

        issue_idx(0, 0)
        wait_idx(0)
        issue_gather(0)

        @pl.when(n_chunks > 1)
        def _():
            issue_idx(1, 1)

        def chunk(i, cc):
            cidx = s + i * _NS
            sl = lax.rem(i, 2)

            def do_slot(sl_s):
                srcb_s = srcs[sl_s]
                dstb_s = dsts[sl_s]
                alb_s = albs[sl_s]
                rows_s = rowss[sl_s]
                wait_gather(sl_s)

                @pl.when(i + 1 < n_chunks)
                def _():
                    wait_idx(1 - sl_s)
                    issue_gather(1 - sl_s)

                def grp(g, c2):
                    dst_v = dstb_s[pl.ds(g * 16, 16)]
                    av = alb_s[pl.ds(g * 16, 16)]
                    ex = jnp.exp(av - gmax)
                    dg = plsc.load_gather(den4, [dst_v * 4 + h])
                    asm = ex / (dg + 1e-16)
                    asmb[pl.ds(g * 16, 16)] = asm
                    return c2
                lax.fori_loop(0, _KA // 16, grp, 0)
                pltpu.sync_copy(
                    asmb.at[pl.ds(0, _KA)],
                    asm_hbm.at[pl.ds(cidx * (HEADS * _KA) + h * _KA, _KA)])

                def scale(e, c2):
                    a0 = asmb[pl.ds(e, 16)][0]
                    for v in range(4):
                        rows_s[e, pl.ds(v * 16, 16)] = (
                            rows_s[e, pl.ds(v * 16, 16)] * a0)
                    return c2
                lax.fori_loop(0, _KA, scale, 0)
                pltpu.sync_copy(rows_s, accsh.at[dstb_s], add=True)

                @pl.when(i + 2 < n_chunks)
                def _():
                    issue_idx(i + 2, sl_s)

            @pl.when(sl == 0)
            def _():
                do_slot(0)

            @pl.when(sl == 1)
            def _():
                do_slot(1)
            return cc
        lax.fori_loop(0, n_chunks, chunk, 0)

    for p in range(2):
        zero_acc()
        plsc.subcore_barrier()

        @pl.when(c == 0)
        def _(_p=p):
            edge_loop(xqs[2 * _p], 2 * _p)

        @pl.when(c == 1)
        def _(_p=p):
            edge_loop(xqs[2 * _p + 1], 2 * _p + 1)

        plsc.subcore_barrier()

        @pl.when(c == 0)
        def _(_p=p):
            pltpu.sync_copy(accsh.at[pl.ds(s * rows_per, rows_per)],
                            accs[2 * _p].at[pl.ds(s * rows_per, rows_per)])

        @pl.when(c == 1)
        def _(_p=p):
            pltpu.sync_copy(accsh.at[pl.ds(s * rows_per, rows_per)],
                            accs[2 * _p + 1].at[pl.ds(s * rows_per,
                                                      rows_per)])

        plsc.subcore_barrier()


def _pass_c(alpha_f, src, dst, wmax, den_p, xq0, xq1, xq2, xq3):
    mesh = plsc.VectorSubcoreMesh(core_axis_name="c", subcore_axis_name="s",
                                  num_cores=_NC, num_subcores=_NS)
    f = pl.kernel(
        _pc_body,
        out_type=[
            jax.ShapeDtypeStruct((E * HEADS,), jnp.float32),
            jax.ShapeDtypeStruct((N, CH), jnp.float32),
            jax.ShapeDtypeStruct((N, CH), jnp.float32),
            jax.ShapeDtypeStruct((N, CH), jnp.float32),
            jax.ShapeDtypeStruct((N, CH), jnp.float32),
        ],
        mesh=mesh,
        scratch_types=[
            pltpu.VMEM((_NW * HEADS * 16,), jnp.float32),
            pltpu.VMEM((_DENP,), jnp.float32),
            pltpu.VMEM((_DSL,), jnp.float32),
            pltpu.VMEM((_KA,), jnp.float32),
            pltpu.VMEM((_KA,), jnp.float32),
            pltpu.VMEM((_KA + 16,), jnp.float32),
            pltpu.VMEM((_KA,), jnp.int32),
            pltpu.VMEM((_KA,), jnp.int32),
            pltpu.VMEM((_KA,), jnp.int32),
            pltpu.VMEM((_KA,), jnp.int32),
            pltpu.VMEM((_KA, CH), jnp.float32),
            pltpu.VMEM((_KA, CH), jnp.float32),
            pltpu.VMEM_SHARED((N, CH), jnp.float32),
            pltpu.SemaphoreType.DMA,
            pltpu.SemaphoreType.DMA,
            pltpu.SemaphoreType.DMA,
            pltpu.SemaphoreType.DMA,
            pltpu.SemaphoreType.DMA,
            pltpu.SemaphoreType.DMA,
            pltpu.SemaphoreType.DMA,
            pltpu.SemaphoreType.DMA,
        ],
        compiler_params=pltpu.CompilerParams(needs_layout_passes=False,
                                             use_tc_tiling_on_sc=False),
    )
    return f(alpha_f, src, dst, wmax, den_p, xq0, xq1, xq2, xq3)


def kernel(x, edge_index, kpts, pts_3d, W_pe1, b_pe1, W_pe2, b_pe2, W_res,
           b_res, W_l, b_l, W_r, b_r, W_e, att, b_conv, ln_g, ln_b, W_proj,
           b_proj):
    x_l, x_r, identity, norm_uv, xq0, xq1, xq2, xq3 = _front(
        x, kpts, pts_3d, W_pe1, b_pe1, W_pe2, b_pe2, W_res, b_res,
        W_l, b_l, W_r, b_r)

    src = edge_index[0]
    dst = edge_index[1]
    alpha_f, ea_f, wmax = _pass_a(x_l, x_r, norm_uv.reshape(-1), src, dst,
                                  W_e.reshape(-1), att.reshape(-1))
    nch = E // _KA
    edge_attr = ea_f.reshape(nch, 3, _KA).transpose(0, 2, 1).reshape(E, 3)

    den_p = _pass_b(alpha_f, dst, wmax)
    asm_f, a0, a1, a2, a3 = _pass_c(alpha_f, src, dst, wmax, den_p,
                                    xq0, xq1, xq2, xq3)
    alpha_sm = asm_f.reshape(nch, HEADS, _KA).transpose(0, 2, 1).reshape(
        E, HEADS)
    acc = jnp.concatenate([a0, a1, a2, a3], axis=1)

    out = _tail(acc, identity, b_conv, ln_g, ln_b, W_proj, b_proj)
    return (out, alpha_sm, edge_attr)
